# baseline (speedup 1.0000x reference)
import functools

import jax
import jax.numpy as jnp
from jax.experimental import pallas as pl
from jax.experimental.pallas import tpu as pltpu

_N = 10000
_E = 320000
_D = 128
_B = 64
_CH = 4
_CUTOFF = 10.0
_BLK = 400


def _node_mlp_body(agg_ref, w1_ref, b1_ref, w2_ref, b2_ref, out_ref):
    acc = None
    for c in range(_CH):
        x = agg_ref[c]
        h1 = jnp.maximum(
            jnp.dot(x, w1_ref[c], preferred_element_type=jnp.float32) + b1_ref[c],
            0.0,
        )
        y = jnp.dot(h1, w2_ref[c], preferred_element_type=jnp.float32) + b2_ref[c]
        acc = y if acc is None else acc + y
    out_ref[...] = acc


@jax.jit
def _node_mlp(agg, w1, b1, w2, b2):
    return pl.pallas_call(
        _node_mlp_body,
        grid=(_N // _BLK,),
        in_specs=[
            pl.BlockSpec((_CH, _BLK, _D), lambda i: (0, i, 0)),
            pl.BlockSpec((_CH, _D, _D), lambda i: (0, 0, 0)),
            pl.BlockSpec((_CH, 1, _D), lambda i: (0, 0, 0)),
            pl.BlockSpec((_CH, _D, _D), lambda i: (0, 0, 0)),
            pl.BlockSpec((_CH, 1, _D), lambda i: (0, 0, 0)),
        ],
        out_specs=pl.BlockSpec((_BLK, _D), lambda i: (i, 0)),
        out_shape=jax.ShapeDtypeStruct((_N, _D), jnp.float32),
    )(agg, w1, b1, w2, b2)


def _mlp(params, x):
    n = len(params)
    for i, (w, b) in enumerate(params):
        x = x @ w + b
        if i < n - 1:
            x = jax.nn.relu(x)
    return x


def kernel(pos, batch, atom_type, edge_index, params):
    src = edge_index[0]
    dst = edge_index[1]
    diff = pos[src] - pos[dst]
    dist = jnp.sqrt(jnp.sum(diff * diff, axis=-1) + 1e-12)
    ea = dist[:, None]
    mu = jnp.mean(ea, axis=0)
    var = jnp.var(ea, axis=0)
    edge_attr = (ea - mu) / jnp.sqrt(var + 1e-5) * params['bn_g'] + params['bn_b']
    edge_weight = (_CUTOFF - ea) / _CUTOFF
    h = params['emb'][atom_type]
    ones = jnp.ones((_N, 1), jnp.float32)
    cnt = jax.ops.segment_sum(ones, batch, num_segments=_B)
    cnt = jnp.maximum(cnt, 1.0)
    n_layers = len(params['layers'])
    for li in range(n_layers):
        lp = params['layers'][li]
        e = jax.nn.softmax(_mlp(lp['edge_mlp'], edge_attr), axis=-1)
        e = e * edge_weight
        hs = h[src]
        aggs = []
        for c in range(_CH):
            msg = hs * e[:, c][:, None]
            aggs.append(jax.ops.segment_sum(msg, dst, num_segments=_N))
        agg = jnp.stack(aggs)
        w1 = jnp.stack([lp['node_mlp'][c][0][0] for c in range(_CH)])
        b1 = jnp.stack([lp['node_mlp'][c][0][1] for c in range(_CH)])[:, None, :]
        w2 = jnp.stack([lp['node_mlp'][c][1][0] for c in range(_CH)])
        b2 = jnp.stack([lp['node_mlp'][c][1][1] for c in range(_CH)])[:, None, :]
        h = _node_mlp(agg, w1, b1, w2, b2)
        if li + 1 < n_layers:
            gn = params['norms'][li]
            mean = jax.ops.segment_sum(h, batch, num_segments=_B) / cnt
            hc = h - mean[batch] * gn['mean_scale']
            v = jax.ops.segment_sum(hc * hc, batch, num_segments=_B) / cnt
            h = hc / jnp.sqrt(v[batch] + 1e-5) * gn['weight'] + gn['bias']
            h = jnp.tanh(h)
    g = jax.ops.segment_sum(h, batch, num_segments=_B) / cnt
    return g.mean(-1)


# speedup vs baseline: 1.2620x; 1.2620x over previous
import functools

import jax
import jax.numpy as jnp
from jax import lax
from jax.experimental import pallas as pl
from jax.experimental.pallas import tpu as pltpu
from jax.experimental.pallas import tpu_sc as plsc

_N = 10000
_E = 320000
_D = 128
_B = 64
_CH = 4
_CUTOFF = 10.0
_BLK = 400

_NC, _NS, _L = 2, 16, 16
_NW = _NC * _NS
_EPW = _E // _NW
_K = 128
_NCHUNK = 79
_EPW_P = _NCHUNK * _K


def _row_range(s):
    base = s * 624
    n16 = jnp.where(s == _NS - 1, 40, 39)
    return base, n16


def _sc_layer_kernel(h_hbm, src_hbm, dst_hbm, e0_hbm, e1_hbm, e2_hbm, e3_hbm,
                     out_hbm,
                     idx_v, dst_v, ev, rows_v,
                     zbuf, stage, acc_sh, sem):
    core = lax.axis_index("c")
    sub = lax.axis_index("s")
    wid = sub * _NC + core
    ebase = wid * _EPW_P
    rowbase, n16 = _row_range(sub)

    zero16 = jnp.zeros((_L,), jnp.float32)
    for r in range(16):
        for j in range(_D // _L):
            zbuf[r, pl.ds(j * _L, _L)] = zero16

    def scale_rows(rows, evv):
        def body(i, _):
            es = evv[i]
            for j in range(_D // _L):
                rows[i, pl.ds(j * _L, _L)] = rows[i, pl.ds(j * _L, _L)] * es
            return 0
        lax.fori_loop(0, _K, body, 0)

    for ch, e_hbm in enumerate((e0_hbm, e1_hbm, e2_hbm, e3_hbm)):
        def zbody(r, _):
            pltpu.sync_copy(zbuf, acc_sh.at[pl.ds(rowbase + r * 16, 16)])
            return 0
        lax.fori_loop(0, n16, zbody, 0)
        plsc.subcore_barrier()

        def chunk(i, _):
            cb = ebase + i * _K
            pltpu.sync_copy(src_hbm.at[pl.ds(cb, _K)], idx_v)
            pltpu.sync_copy(dst_hbm.at[pl.ds(cb, _K)], dst_v)
            pltpu.sync_copy(e_hbm.at[pl.ds(cb, _K)], ev)
            pltpu.async_copy(h_hbm.at[idx_v], rows_v, sem).wait()
            scale_rows(rows_v, ev)
            pltpu.sync_copy(rows_v, acc_sh.at[dst_v], add=True)
            return 0
        lax.fori_loop(0, _NCHUNK, chunk, 0)

        plsc.subcore_barrier()

        def rbody(r, _):
            pltpu.sync_copy(acc_sh.at[pl.ds(rowbase + r * 16, 16)], stage)
            pltpu.sync_copy(stage, out_hbm.at[core, ch,
                                             pl.ds(rowbase + r * 16, 16)])
            return 0
        lax.fori_loop(0, n16, rbody, 0)


@jax.jit
def _sc_layer(h, src, dst, e0, e1, e2, e3):
    mesh = plsc.VectorSubcoreMesh(core_axis_name="c", subcore_axis_name="s")
    f = functools.partial(
        pl.kernel,
        mesh=mesh,
        out_type=jax.ShapeDtypeStruct((_NC, _CH, _N, _D), jnp.float32),
        scratch_types=[
            pltpu.VMEM((_K,), jnp.int32),
            pltpu.VMEM((_K,), jnp.int32),
            pltpu.VMEM((_K, _L), jnp.float32),
            pltpu.VMEM((_K, _D), jnp.float32),
            pltpu.VMEM((16, _D), jnp.float32),
            pltpu.VMEM((16, _D), jnp.float32),
            pltpu.VMEM_SHARED((_N, _D), jnp.float32),
            pltpu.SemaphoreType.DMA,
        ],
    )(_sc_layer_kernel)
    return f(h, src, dst, e0, e1, e2, e3)


def _node_mlp_body(agg_ref, w1_ref, b1_ref, w2_ref, b2_ref, out_ref):
    acc = None
    for c in range(_CH):
        x = agg_ref[0, c] + agg_ref[1, c]
        h1 = jnp.maximum(
            jnp.dot(x, w1_ref[c], preferred_element_type=jnp.float32) + b1_ref[c],
            0.0,
        )
        y = jnp.dot(h1, w2_ref[c], preferred_element_type=jnp.float32) + b2_ref[c]
        acc = y if acc is None else acc + y
    out_ref[...] = acc


@jax.jit
def _node_mlp(agg, w1, b1, w2, b2):
    return pl.pallas_call(
        _node_mlp_body,
        grid=(_N // _BLK,),
        in_specs=[
            pl.BlockSpec((_NC, _CH, _BLK, _D), lambda i: (0, 0, i, 0)),
            pl.BlockSpec((_CH, _D, _D), lambda i: (0, 0, 0)),
            pl.BlockSpec((_CH, 1, _D), lambda i: (0, 0, 0)),
            pl.BlockSpec((_CH, _D, _D), lambda i: (0, 0, 0)),
            pl.BlockSpec((_CH, 1, _D), lambda i: (0, 0, 0)),
        ],
        out_specs=pl.BlockSpec((_BLK, _D), lambda i: (i, 0)),
        out_shape=jax.ShapeDtypeStruct((_N, _D), jnp.float32),
    )(agg, w1, b1, w2, b2)


def _mlp(params, x):
    n = len(params)
    for i, (w, b) in enumerate(params):
        x = x @ w + b
        if i < n - 1:
            x = jax.nn.relu(x)
    return x


def kernel(pos, batch, atom_type, edge_index, params):
    src = edge_index[0]
    dst = edge_index[1]
    diff = pos[src] - pos[dst]
    dist = jnp.sqrt(jnp.sum(diff * diff, axis=-1) + 1e-12)
    ea = dist[:, None]
    mu = jnp.mean(ea, axis=0)
    var = jnp.var(ea, axis=0)
    edge_attr = (ea - mu) / jnp.sqrt(var + 1e-5) * params['bn_g'] + params['bn_b']
    edge_weight = (_CUTOFF - ea) / _CUTOFF
    h = params['emb'][atom_type]
    ones = jnp.ones((_N, 1), jnp.float32)
    cnt = jax.ops.segment_sum(ones, batch, num_segments=_B)
    cnt = jnp.maximum(cnt, 1.0)
    n_layers = len(params['layers'])
    for li in range(n_layers):
        lp = params['layers'][li]
        e = jax.nn.softmax(_mlp(lp['edge_mlp'], edge_attr), axis=-1)
        e = e * edge_weight
        pad = _EPW_P - _EPW
        ee = [jnp.pad(jnp.broadcast_to(e[:, c][:, None], (_E, _L))
                      .reshape(_NW, _EPW, _L), ((0, 0), (0, pad), (0, 0)))
              .reshape(_NW * _EPW_P, _L) for c in range(_CH)]
        src_p = jnp.pad(src.reshape(_NW, _EPW), ((0, 0), (0, pad))).reshape(-1)
        dst_p = jnp.pad(dst.reshape(_NW, _EPW), ((0, 0), (0, pad))).reshape(-1)
        agg = _sc_layer(h, src_p, dst_p, ee[0], ee[1], ee[2], ee[3])
        w1 = jnp.stack([lp['node_mlp'][c][0][0] for c in range(_CH)])
        b1 = jnp.stack([lp['node_mlp'][c][0][1] for c in range(_CH)])[:, None, :]
        w2 = jnp.stack([lp['node_mlp'][c][1][0] for c in range(_CH)])
        b2 = jnp.stack([lp['node_mlp'][c][1][1] for c in range(_CH)])[:, None, :]
        h = _node_mlp(agg, w1, b1, w2, b2)
        if li + 1 < n_layers:
            gn = params['norms'][li]
            mean = jax.ops.segment_sum(h, batch, num_segments=_B) / cnt
            hc = h - mean[batch] * gn['mean_scale']
            v = jax.ops.segment_sum(hc * hc, batch, num_segments=_B) / cnt
            h = hc / jnp.sqrt(v[batch] + 1e-5) * gn['weight'] + gn['bias']
            h = jnp.tanh(h)
    g = jax.ops.segment_sum(h, batch, num_segments=_B) / cnt
    return g.mean(-1)


# speedup vs baseline: 1.5648x; 1.2399x over previous
import functools

import jax
import jax.numpy as jnp
from jax import lax
from jax.experimental import pallas as pl
from jax.experimental.pallas import tpu as pltpu
from jax.experimental.pallas import tpu_sc as plsc

_N = 10000
_E = 320000
_D = 128
_B = 64
_CH = 4
_CUTOFF = 10.0
_BLK = 400

_NC, _NS, _L = 2, 16, 16
_NW = _NC * _NS
_EPW = _E // _NW
_K = 64
_NCHUNK = 158
_EPW_P = _NCHUNK * _K


def _row_range(s):
    base = s * 624
    n16 = jnp.where(s == _NS - 1, 40, 39)
    return base, n16


def _sc_layer_kernel(h_hbm, src_hbm, dst_hbm, e0_hbm, e1_hbm, e2_hbm, e3_hbm,
                     out_hbm,
                     idx_a, idx_b, dst_a, dst_b, ev_a, ev_b, rows_a, rows_b,
                     zbuf,
                     sem_a0, sem_a1, sem_g0, sem_g1, acc_sh):
    core = lax.axis_index("c")
    sub = lax.axis_index("s")
    wid = sub * _NC + core
    ebase = wid * _EPW_P
    rowbase, n16 = _row_range(sub)

    idx = (idx_a, idx_b)
    dstv = (dst_a, dst_b)
    ev = (ev_a, ev_b)
    rows = (rows_a, rows_b)
    sem_as = (sem_a0, sem_a1)
    sem_gs = (sem_g0, sem_g1)

    def scale_rows(rw, evv):
        def body(i, _):
            es = evv[i]
            for j in range(_D // _L):
                rw[i, pl.ds(j * _L, _L)] = rw[i, pl.ds(j * _L, _L)] * es
            return 0
        lax.fori_loop(0, _K, body, 0)

    for ch, e_hbm in enumerate((e0_hbm, e1_hbm, e2_hbm, e3_hbm)):
        def issue_small(c, b):
            cb = ebase + c * _K
            pltpu.async_copy(src_hbm.at[pl.ds(cb, _K)], idx[b], sem_as[b])
            pltpu.async_copy(dst_hbm.at[pl.ds(cb, _K)], dstv[b], sem_as[b])
            pltpu.async_copy(e_hbm.at[pl.ds(cb, _K)], ev[b], sem_as[b])

        def wait_small(b):
            pltpu.make_async_copy(src_hbm.at[pl.ds(0, _K)], idx[b],
                                  sem_as[b]).wait()
            pltpu.make_async_copy(dst_hbm.at[pl.ds(0, _K)], dstv[b],
                                  sem_as[b]).wait()
            pltpu.make_async_copy(e_hbm.at[pl.ds(0, _K)], ev[b],
                                  sem_as[b]).wait()

        def issue_gather(b):
            pltpu.async_copy(h_hbm.at[idx[b]], rows[b], sem_gs[b])

        def wait_gather(b):
            pltpu.make_async_copy(h_hbm.at[pl.ds(0, _K)], rows[b],
                                  sem_gs[b]).wait()

        def finish(b):
            wait_gather(b)
            scale_rows(rows[b], ev[b])
            pltpu.sync_copy(rows[b], acc_sh.at[dstv[b]], add=True)

        zero16 = jnp.zeros((_L,), jnp.float32)
        for r in range(16):
            for j in range(_D // _L):
                zbuf[r, pl.ds(j * _L, _L)] = zero16

        def zbody(r, _):
            pltpu.sync_copy(zbuf, acc_sh.at[pl.ds(rowbase + r * 16, 16)])
            return 0
        lax.fori_loop(0, n16, zbody, 0)
        plsc.subcore_barrier()

        issue_small(0, 0)
        wait_small(0)
        issue_gather(0)
        issue_small(1, 1)

        def step(c, cur, oth):
            wait_small(cur)
            issue_gather(cur)
            finish(oth)
            issue_small(jnp.minimum(c + 1, _NCHUNK - 1), oth)

        def pair(p, _):
            step(2 * p + 1, 1, 0)
            step(2 * p + 2, 0, 1)
            return 0
        lax.fori_loop(0, (_NCHUNK - 1) // 2, pair, 0)
        wait_small(1)
        finish(0)

        plsc.subcore_barrier()

        def rbody(r, _):
            pltpu.sync_copy(acc_sh.at[pl.ds(rowbase + r * 16, 16)], zbuf)
            pltpu.sync_copy(zbuf, out_hbm.at[core, ch,
                                            pl.ds(rowbase + r * 16, 16)])
            return 0
        lax.fori_loop(0, n16, rbody, 0)


@jax.jit
def _sc_layer(h, src, dst, e0, e1, e2, e3):
    mesh = plsc.VectorSubcoreMesh(core_axis_name="c", subcore_axis_name="s")
    f = functools.partial(
        pl.kernel,
        mesh=mesh,
        out_type=jax.ShapeDtypeStruct((_NC, _CH, _N, _D), jnp.float32),
        scratch_types=[
            pltpu.VMEM((_K,), jnp.int32),
            pltpu.VMEM((_K,), jnp.int32),
            pltpu.VMEM((_K,), jnp.int32),
            pltpu.VMEM((_K,), jnp.int32),
            pltpu.VMEM((_K, _L), jnp.float32),
            pltpu.VMEM((_K, _L), jnp.float32),
            pltpu.VMEM((_K, _D), jnp.float32),
            pltpu.VMEM((_K, _D), jnp.float32),
            pltpu.VMEM((16, _D), jnp.float32),
            pltpu.SemaphoreType.DMA,
            pltpu.SemaphoreType.DMA,
            pltpu.SemaphoreType.DMA,
            pltpu.SemaphoreType.DMA,
            pltpu.VMEM_SHARED((_N, _D), jnp.float32),
        ],
    )(_sc_layer_kernel)
    return f(h, src, dst, e0, e1, e2, e3)


_NPW = 320
_EK = 80


def _sc_emb_kernel(emb_hbm, at_hbm, out_hbm, idxv, rowsv, sem):
    core = lax.axis_index("c")
    sub = lax.axis_index("s")
    wid = sub * _NC + core
    base = wid * _NPW
    for c in range(_NPW // _EK):
        pltpu.sync_copy(at_hbm.at[pl.ds(base + c * _EK, _EK)], idxv)
        pltpu.async_copy(emb_hbm.at[idxv], rowsv, sem).wait()
        pltpu.sync_copy(rowsv, out_hbm.at[pl.ds(base + c * _EK, _EK)])


@jax.jit
def _sc_emb(emb, at_p):
    mesh = plsc.VectorSubcoreMesh(core_axis_name="c", subcore_axis_name="s")
    f = functools.partial(
        pl.kernel,
        mesh=mesh,
        out_type=jax.ShapeDtypeStruct((_NW * _NPW, _D), jnp.float32),
        scratch_types=[
            pltpu.VMEM((_EK,), jnp.int32),
            pltpu.VMEM((_EK, _D), jnp.float32),
            pltpu.SemaphoreType.DMA,
        ],
    )(_sc_emb_kernel)
    return f(emb, at_p)


def _node_mlp_body(agg_ref, w1_ref, b1_ref, w2_ref, b2_ref, out_ref):
    acc = None
    for c in range(_CH):
        x = agg_ref[0, c] + agg_ref[1, c]
        h1 = jnp.maximum(
            jnp.dot(x, w1_ref[c], preferred_element_type=jnp.float32) + b1_ref[c],
            0.0,
        )
        y = jnp.dot(h1, w2_ref[c], preferred_element_type=jnp.float32) + b2_ref[c]
        acc = y if acc is None else acc + y
    out_ref[...] = acc


@jax.jit
def _node_mlp(agg, w1, b1, w2, b2):
    return pl.pallas_call(
        _node_mlp_body,
        grid=(_N // _BLK,),
        in_specs=[
            pl.BlockSpec((_NC, _CH, _BLK, _D), lambda i: (0, 0, i, 0)),
            pl.BlockSpec((_CH, _D, _D), lambda i: (0, 0, 0)),
            pl.BlockSpec((_CH, 1, _D), lambda i: (0, 0, 0)),
            pl.BlockSpec((_CH, _D, _D), lambda i: (0, 0, 0)),
            pl.BlockSpec((_CH, 1, _D), lambda i: (0, 0, 0)),
        ],
        out_specs=pl.BlockSpec((_BLK, _D), lambda i: (i, 0)),
        out_shape=jax.ShapeDtypeStruct((_N, _D), jnp.float32),
    )(agg, w1, b1, w2, b2)


def _mlp(params, x):
    n = len(params)
    for i, (w, b) in enumerate(params):
        x = x @ w + b
        if i < n - 1:
            x = jax.nn.relu(x)
    return x


def kernel(pos, batch, atom_type, edge_index, params):
    src = edge_index[0]
    dst = edge_index[1]
    diff = pos[src] - pos[dst]
    dist = jnp.sqrt(jnp.sum(diff * diff, axis=-1) + 1e-12)
    ea = dist[:, None]
    mu = jnp.mean(ea, axis=0)
    var = jnp.var(ea, axis=0)
    edge_attr = (ea - mu) / jnp.sqrt(var + 1e-5) * params['bn_g'] + params['bn_b']
    edge_weight = (_CUTOFF - ea) / _CUTOFF
    at_p = jnp.pad(atom_type, (0, _NW * _NPW - _N))
    h = _sc_emb(params['emb'], at_p)[:_N]
    ones = jnp.ones((_N, 1), jnp.float32)
    cnt = jax.ops.segment_sum(ones, batch, num_segments=_B)
    cnt = jnp.maximum(cnt, 1.0)
    n_layers = len(params['layers'])
    for li in range(n_layers):
        lp = params['layers'][li]
        e = jax.nn.softmax(_mlp(lp['edge_mlp'], edge_attr), axis=-1)
        e = e * edge_weight
        pad = _EPW_P - _EPW
        ee = [jnp.pad(jnp.broadcast_to(e[:, c][:, None], (_E, _L))
                      .reshape(_NW, _EPW, _L), ((0, 0), (0, pad), (0, 0)))
              .reshape(_NW * _EPW_P, _L) for c in range(_CH)]
        src_p = jnp.pad(src.reshape(_NW, _EPW), ((0, 0), (0, pad))).reshape(-1)
        dst_p = jnp.pad(dst.reshape(_NW, _EPW), ((0, 0), (0, pad))).reshape(-1)
        agg = _sc_layer(h, src_p, dst_p, ee[0], ee[1], ee[2], ee[3])
        w1 = jnp.stack([lp['node_mlp'][c][0][0] for c in range(_CH)])
        b1 = jnp.stack([lp['node_mlp'][c][0][1] for c in range(_CH)])[:, None, :]
        w2 = jnp.stack([lp['node_mlp'][c][1][0] for c in range(_CH)])
        b2 = jnp.stack([lp['node_mlp'][c][1][1] for c in range(_CH)])[:, None, :]
        h = _node_mlp(agg, w1, b1, w2, b2)
        if li + 1 < n_layers:
            gn = params['norms'][li]
            mean = jax.ops.segment_sum(h, batch, num_segments=_B) / cnt
            hc = h - mean[batch] * gn['mean_scale']
            v = jax.ops.segment_sum(hc * hc, batch, num_segments=_B) / cnt
            h = hc / jnp.sqrt(v[batch] + 1e-5) * gn['weight'] + gn['bias']
            h = jnp.tanh(h)
    g = jax.ops.segment_sum(h, batch, num_segments=_B) / cnt
    return g.mean(-1)


# speedup vs baseline: 1.6696x; 1.0670x over previous
import functools

import jax
import jax.numpy as jnp
from jax import lax
from jax.experimental import pallas as pl
from jax.experimental.pallas import tpu as pltpu
from jax.experimental.pallas import tpu_sc as plsc

_N = 10000
_E = 320000
_D = 128
_B = 64
_CH = 4
_CUTOFF = 10.0
_BLK = 400

_NC, _NS, _L = 2, 16, 16
_NW = _NC * _NS
_EPW = _E // _NW
_K = 64
_NCHUNK = 158
_EPW_P = _NCHUNK * _K


def _row_range(s):
    base = s * 624
    n16 = jnp.where(s == _NS - 1, 40, 39)
    return base, n16


def _sc_layer_kernel(h_hbm, src_hbm, dst_hbm, e0_hbm, e1_hbm, e2_hbm, e3_hbm,
                     out_hbm,
                     idx_a, idx_b, dst_a, dst_b, ev_a, ev_b, rows_a, rows_b,
                     zbuf,
                     sem_a0, sem_a1, sem_g0, sem_g1, acc_sh):
    core = lax.axis_index("c")
    sub = lax.axis_index("s")
    wid = sub * _NC + core
    ebase = wid * _EPW_P
    rowbase, n16 = _row_range(sub)

    idx = (idx_a, idx_b)
    dstv = (dst_a, dst_b)
    ev = (ev_a, ev_b)
    rows = (rows_a, rows_b)
    sem_as = (sem_a0, sem_a1)
    sem_gs = (sem_g0, sem_g1)

    def scale_rows(rw, evv):
        def body(i, _):
            es = evv[i]
            for j in range(_D // _L):
                rw[i, pl.ds(j * _L, _L)] = rw[i, pl.ds(j * _L, _L)] * es
            return 0
        lax.fori_loop(0, _K, body, 0)

    for ch, e_hbm in enumerate((e0_hbm, e1_hbm, e2_hbm, e3_hbm)):
        def issue_small(c, b):
            cb = ebase + c * _K
            pltpu.async_copy(src_hbm.at[pl.ds(cb, _K)], idx[b], sem_as[b])
            pltpu.async_copy(dst_hbm.at[pl.ds(cb, _K)], dstv[b], sem_as[b])
            pltpu.async_copy(e_hbm.at[pl.ds(cb, _K)], ev[b], sem_as[b])

        def wait_small(b):
            pltpu.make_async_copy(src_hbm.at[pl.ds(0, _K)], idx[b],
                                  sem_as[b]).wait()
            pltpu.make_async_copy(dst_hbm.at[pl.ds(0, _K)], dstv[b],
                                  sem_as[b]).wait()
            pltpu.make_async_copy(e_hbm.at[pl.ds(0, _K)], ev[b],
                                  sem_as[b]).wait()

        def issue_gather(b):
            pltpu.async_copy(h_hbm.at[idx[b]], rows[b], sem_gs[b])

        def wait_gather(b):
            pltpu.make_async_copy(h_hbm.at[pl.ds(0, _K)], rows[b],
                                  sem_gs[b]).wait()

        def finish(b):
            wait_gather(b)
            scale_rows(rows[b], ev[b])
            pltpu.sync_copy(rows[b], acc_sh.at[dstv[b]], add=True)

        zero16 = jnp.zeros((_L,), jnp.float32)
        for r in range(16):
            for j in range(_D // _L):
                zbuf[r, pl.ds(j * _L, _L)] = zero16

        def zbody(r, _):
            pltpu.sync_copy(zbuf, acc_sh.at[pl.ds(rowbase + r * 16, 16)])
            return 0
        lax.fori_loop(0, n16, zbody, 0)
        plsc.subcore_barrier()

        issue_small(0, 0)
        wait_small(0)
        issue_gather(0)
        issue_small(1, 1)

        def step(c, cur, oth):
            wait_small(cur)
            issue_gather(cur)
            finish(oth)
            issue_small(jnp.minimum(c + 1, _NCHUNK - 1), oth)

        def pair(p, _):
            step(2 * p + 1, 1, 0)
            step(2 * p + 2, 0, 1)
            return 0
        lax.fori_loop(0, (_NCHUNK - 1) // 2, pair, 0)
        wait_small(1)
        finish(0)

        plsc.subcore_barrier()

        def rbody(r, _):
            pltpu.sync_copy(acc_sh.at[pl.ds(rowbase + r * 16, 16)], zbuf)
            pltpu.sync_copy(zbuf, out_hbm.at[core, ch,
                                            pl.ds(rowbase + r * 16, 16)])
            return 0
        lax.fori_loop(0, n16, rbody, 0)


@jax.jit
def _sc_layer(h, src, dst, e0, e1, e2, e3):
    mesh = plsc.VectorSubcoreMesh(core_axis_name="c", subcore_axis_name="s")
    f = functools.partial(
        pl.kernel,
        mesh=mesh,
        out_type=jax.ShapeDtypeStruct((_NC, _CH, _N, _D), jnp.float32),
        scratch_types=[
            pltpu.VMEM((_K,), jnp.int32),
            pltpu.VMEM((_K,), jnp.int32),
            pltpu.VMEM((_K,), jnp.int32),
            pltpu.VMEM((_K,), jnp.int32),
            pltpu.VMEM((_K, _L), jnp.float32),
            pltpu.VMEM((_K, _L), jnp.float32),
            pltpu.VMEM((_K, _D), jnp.float32),
            pltpu.VMEM((_K, _D), jnp.float32),
            pltpu.VMEM((16, _D), jnp.float32),
            pltpu.SemaphoreType.DMA,
            pltpu.SemaphoreType.DMA,
            pltpu.SemaphoreType.DMA,
            pltpu.SemaphoreType.DMA,
            pltpu.VMEM_SHARED((_N, _D), jnp.float32),
        ],
    )(_sc_layer_kernel)
    return f(h, src, dst, e0, e1, e2, e3)


_GK = 128
_GNCH = _EPW_P // _GK


def _sc_geom_kernel(pos_hbm, src_hbm, dst_hbm, out_hbm,
                    si_a, si_b, di_a, di_b, pa_a, pa_b, pb_a, pb_b,
                    ow_a, ow_b, sem_a0, sem_a1, sem_g0, sem_g1):
    core = lax.axis_index("c")
    sub = lax.axis_index("s")
    wid = sub * _NC + core
    ebase = wid * _EPW_P

    si = (si_a, si_b)
    di = (di_a, di_b)
    pa = (pa_a, pa_b)
    pb = (pb_a, pb_b)
    ow = (ow_a, ow_b)
    sem_as = (sem_a0, sem_a1)
    sem_gs = (sem_g0, sem_g1)

    def issue_small(c, b):
        cb = ebase + c * _GK
        pltpu.async_copy(src_hbm.at[pl.ds(cb, _GK)], si[b], sem_as[b])
        pltpu.async_copy(dst_hbm.at[pl.ds(cb, _GK)], di[b], sem_as[b])

    def wait_small(b):
        pltpu.make_async_copy(src_hbm.at[pl.ds(0, _GK)], si[b],
                              sem_as[b]).wait()
        pltpu.make_async_copy(src_hbm.at[pl.ds(0, _GK)], di[b],
                              sem_as[b]).wait()

    def issue_gather(b):
        pltpu.async_copy(pos_hbm.at[si[b]], pa[b], sem_gs[b])
        pltpu.async_copy(pos_hbm.at[di[b]], pb[b], sem_gs[b])

    def wait_gather(b):
        pltpu.make_async_copy(pos_hbm.at[pl.ds(0, _GK)], pa[b],
                              sem_gs[b]).wait()
        pltpu.make_async_copy(pos_hbm.at[pl.ds(0, _GK)], pb[b],
                              sem_gs[b]).wait()

    def finish(c, b):
        wait_gather(b)

        def body(i, _):
            d = pa[b][i, pl.ds(0, _L)] - pb[b][i, pl.ds(0, _L)]
            ow[b][i] = d * d
            return 0
        lax.fori_loop(0, _GK, body, 0)
        pltpu.sync_copy(ow[b], out_hbm.at[pl.ds(ebase + c * _GK, _GK)])

    issue_small(0, 0)
    wait_small(0)
    issue_gather(0)
    issue_small(1, 1)

    def step(c, cur, oth):
        wait_small(cur)
        issue_gather(cur)
        finish(c - 1, oth)
        issue_small(jnp.minimum(c + 1, _GNCH - 1), oth)

    def pair(p, _):
        step(2 * p + 1, 1, 0)
        step(2 * p + 2, 0, 1)
        return 0
    lax.fori_loop(0, (_GNCH - 1) // 2, pair, 0)
    wait_small(1)
    finish(_GNCH - 1, 0)


@jax.jit
def _sc_geom(pos16, src_p, dst_p):
    mesh = plsc.VectorSubcoreMesh(core_axis_name="c", subcore_axis_name="s")
    f = functools.partial(
        pl.kernel,
        mesh=mesh,
        out_type=jax.ShapeDtypeStruct((_NW * _EPW_P, _L), jnp.float32),
        scratch_types=[
            pltpu.VMEM((_GK,), jnp.int32),
            pltpu.VMEM((_GK,), jnp.int32),
            pltpu.VMEM((_GK,), jnp.int32),
            pltpu.VMEM((_GK,), jnp.int32),
            pltpu.VMEM((_GK, _D), jnp.float32),
            pltpu.VMEM((_GK, _D), jnp.float32),
            pltpu.VMEM((_GK, _D), jnp.float32),
            pltpu.VMEM((_GK, _D), jnp.float32),
            pltpu.VMEM((_GK, _L), jnp.float32),
            pltpu.VMEM((_GK, _L), jnp.float32),
            pltpu.SemaphoreType.DMA,
            pltpu.SemaphoreType.DMA,
            pltpu.SemaphoreType.DMA,
            pltpu.SemaphoreType.DMA,
        ],
    )(_sc_geom_kernel)
    return f(pos16, src_p, dst_p)


_NPW = 320
_EK = 80


def _sc_emb_kernel(emb_hbm, at_hbm, out_hbm, idxv, rowsv, sem):
    core = lax.axis_index("c")
    sub = lax.axis_index("s")
    wid = sub * _NC + core
    base = wid * _NPW
    for c in range(_NPW // _EK):
        pltpu.sync_copy(at_hbm.at[pl.ds(base + c * _EK, _EK)], idxv)
        pltpu.async_copy(emb_hbm.at[idxv], rowsv, sem).wait()
        pltpu.sync_copy(rowsv, out_hbm.at[pl.ds(base + c * _EK, _EK)])


@jax.jit
def _sc_emb(emb, at_p):
    mesh = plsc.VectorSubcoreMesh(core_axis_name="c", subcore_axis_name="s")
    f = functools.partial(
        pl.kernel,
        mesh=mesh,
        out_type=jax.ShapeDtypeStruct((_NW * _NPW, _D), jnp.float32),
        scratch_types=[
            pltpu.VMEM((_EK,), jnp.int32),
            pltpu.VMEM((_EK, _D), jnp.float32),
            pltpu.SemaphoreType.DMA,
        ],
    )(_sc_emb_kernel)
    return f(emb, at_p)


def _node_mlp_body(agg_ref, w1_ref, b1_ref, w2_ref, b2_ref, out_ref):
    acc = None
    for c in range(_CH):
        x = agg_ref[0, c] + agg_ref[1, c]
        h1 = jnp.maximum(
            jnp.dot(x, w1_ref[c], preferred_element_type=jnp.float32) + b1_ref[c],
            0.0,
        )
        y = jnp.dot(h1, w2_ref[c], preferred_element_type=jnp.float32) + b2_ref[c]
        acc = y if acc is None else acc + y
    out_ref[...] = acc


@jax.jit
def _node_mlp(agg, w1, b1, w2, b2):
    return pl.pallas_call(
        _node_mlp_body,
        grid=(_N // _BLK,),
        in_specs=[
            pl.BlockSpec((_NC, _CH, _BLK, _D), lambda i: (0, 0, i, 0)),
            pl.BlockSpec((_CH, _D, _D), lambda i: (0, 0, 0)),
            pl.BlockSpec((_CH, 1, _D), lambda i: (0, 0, 0)),
            pl.BlockSpec((_CH, _D, _D), lambda i: (0, 0, 0)),
            pl.BlockSpec((_CH, 1, _D), lambda i: (0, 0, 0)),
        ],
        out_specs=pl.BlockSpec((_BLK, _D), lambda i: (i, 0)),
        out_shape=jax.ShapeDtypeStruct((_N, _D), jnp.float32),
    )(agg, w1, b1, w2, b2)


def _mlp(params, x):
    n = len(params)
    for i, (w, b) in enumerate(params):
        x = x @ w + b
        if i < n - 1:
            x = jax.nn.relu(x)
    return x


def kernel(pos, batch, atom_type, edge_index, params):
    src = edge_index[0]
    dst = edge_index[1]
    pad = _EPW_P - _EPW
    src_p = jnp.pad(src.reshape(_NW, _EPW), ((0, 0), (0, pad))).reshape(-1)
    dst_p = jnp.pad(dst.reshape(_NW, _EPW), ((0, 0), (0, pad))).reshape(-1)
    pos16 = jnp.pad(pos, ((0, 0), (0, _D - 3)))
    d2w = _sc_geom(pos16, src_p, dst_p).reshape(_NW, _EPW_P, _L)[:, :_EPW, :]
    dist = jnp.sqrt(d2w[..., 0] + d2w[..., 1] + d2w[..., 2]
                    + 1e-12).reshape(_E)
    ea = dist[:, None]
    mu = jnp.mean(ea, axis=0)
    var = jnp.var(ea, axis=0)
    edge_attr = (ea - mu) / jnp.sqrt(var + 1e-5) * params['bn_g'] + params['bn_b']
    edge_weight = (_CUTOFF - ea) / _CUTOFF
    at_p = jnp.pad(atom_type, (0, _NW * _NPW - _N))
    h = _sc_emb(params['emb'], at_p)[:_N]
    ones = jnp.ones((_N, 1), jnp.float32)
    cnt = jax.ops.segment_sum(ones, batch, num_segments=_B)
    cnt = jnp.maximum(cnt, 1.0)
    n_layers = len(params['layers'])
    for li in range(n_layers):
        lp = params['layers'][li]
        e = jax.nn.softmax(_mlp(lp['edge_mlp'], edge_attr), axis=-1)
        e = e * edge_weight
        ee = [jnp.pad(jnp.broadcast_to(e[:, c][:, None], (_E, _L))
                      .reshape(_NW, _EPW, _L), ((0, 0), (0, pad), (0, 0)))
              .reshape(_NW * _EPW_P, _L) for c in range(_CH)]
        agg = _sc_layer(h, src_p, dst_p, ee[0], ee[1], ee[2], ee[3])
        w1 = jnp.stack([lp['node_mlp'][c][0][0] for c in range(_CH)])
        b1 = jnp.stack([lp['node_mlp'][c][0][1] for c in range(_CH)])[:, None, :]
        w2 = jnp.stack([lp['node_mlp'][c][1][0] for c in range(_CH)])
        b2 = jnp.stack([lp['node_mlp'][c][1][1] for c in range(_CH)])[:, None, :]
        h = _node_mlp(agg, w1, b1, w2, b2)
        if li + 1 < n_layers:
            gn = params['norms'][li]
            mean = jax.ops.segment_sum(h, batch, num_segments=_B) / cnt
            hc = h - mean[batch] * gn['mean_scale']
            v = jax.ops.segment_sum(hc * hc, batch, num_segments=_B) / cnt
            h = hc / jnp.sqrt(v[batch] + 1e-5) * gn['weight'] + gn['bias']
            h = jnp.tanh(h)
    g = jax.ops.segment_sum(h, batch, num_segments=_B) / cnt
    return g.mean(-1)


# speedup vs baseline: 1.8941x; 1.1345x over previous
import functools

import jax
import jax.numpy as jnp
from jax import lax
from jax.experimental import pallas as pl
from jax.experimental.pallas import tpu as pltpu
from jax.experimental.pallas import tpu_sc as plsc

_N = 10000
_E = 320000
_D = 128
_B = 64
_CH = 4
_CUTOFF = 10.0
_BLK = 400

_NC, _NS, _L = 2, 16, 16
_NW = _NC * _NS
_EPW = _E // _NW
_K = 64
_NCHUNK = 158
_EPW_P = _NCHUNK * _K


def _row_range(s):
    base = s * 624
    n16 = jnp.where(s == _NS - 1, 40, 39)
    return base, n16


def _sc_layer_kernel(h_hbm, src_hbm, dst_hbm, e0_hbm, e1_hbm, e2_hbm, e3_hbm,
                     out_hbm,
                     idx_a, idx_b, dst_a, dst_b, ev_a, ev_b, rows_a, rows_b,
                     zbuf,
                     sem_a0, sem_a1, sem_g0, sem_g1, acc_sh):
    core = lax.axis_index("c")
    sub = lax.axis_index("s")
    wid = sub * _NC + core
    ebase = wid * _EPW_P
    rowbase, n16 = _row_range(sub)

    idx = (idx_a, idx_b)
    dstv = (dst_a, dst_b)
    ev = (ev_a, ev_b)
    rows = (rows_a, rows_b)
    sem_as = (sem_a0, sem_a1)
    sem_gs = (sem_g0, sem_g1)

    def scale_rows(rw, evv):
        def body(i, _):
            es = evv[i]
            for j in range(_D // _L):
                rw[i, pl.ds(j * _L, _L)] = rw[i, pl.ds(j * _L, _L)] * es
            return 0
        lax.fori_loop(0, _K, body, 0)

    for ch, e_hbm in enumerate((e0_hbm, e1_hbm, e2_hbm, e3_hbm)):
        def issue_small(c, b):
            cb = ebase + c * _K
            pltpu.async_copy(src_hbm.at[pl.ds(cb, _K)], idx[b], sem_as[b])
            pltpu.async_copy(dst_hbm.at[pl.ds(cb, _K)], dstv[b], sem_as[b])
            pltpu.async_copy(e_hbm.at[pl.ds(cb, _K)], ev[b], sem_as[b])

        def wait_small(b):
            pltpu.make_async_copy(src_hbm.at[pl.ds(0, _K)], idx[b],
                                  sem_as[b]).wait()
            pltpu.make_async_copy(dst_hbm.at[pl.ds(0, _K)], dstv[b],
                                  sem_as[b]).wait()
            pltpu.make_async_copy(e_hbm.at[pl.ds(0, _K)], ev[b],
                                  sem_as[b]).wait()

        def issue_gather(b):
            pltpu.async_copy(h_hbm.at[idx[b]], rows[b], sem_gs[b])

        def wait_gather(b):
            pltpu.make_async_copy(h_hbm.at[pl.ds(0, _K)], rows[b],
                                  sem_gs[b]).wait()

        def finish(b):
            wait_gather(b)
            scale_rows(rows[b], ev[b])
            pltpu.sync_copy(rows[b], acc_sh.at[dstv[b]], add=True)

        zero16 = jnp.zeros((_L,), jnp.float32)
        for r in range(16):
            for j in range(_D // _L):
                zbuf[r, pl.ds(j * _L, _L)] = zero16

        def zbody(r, _):
            pltpu.sync_copy(zbuf, acc_sh.at[pl.ds(rowbase + r * 16, 16)])
            return 0
        lax.fori_loop(0, n16, zbody, 0)
        plsc.subcore_barrier()

        issue_small(0, 0)
        wait_small(0)
        issue_gather(0)
        issue_small(1, 1)

        def step(c, cur, oth):
            wait_small(cur)
            issue_gather(cur)
            finish(oth)
            issue_small(jnp.minimum(c + 1, _NCHUNK - 1), oth)

        def pair(p, _):
            step(2 * p + 1, 1, 0)
            step(2 * p + 2, 0, 1)
            return 0
        lax.fori_loop(0, (_NCHUNK - 1) // 2, pair, 0)
        wait_small(1)
        finish(0)

        plsc.subcore_barrier()

        def rbody(r, _):
            pltpu.sync_copy(acc_sh.at[pl.ds(rowbase + r * 16, 16)], zbuf)
            pltpu.sync_copy(zbuf, out_hbm.at[core, ch,
                                            pl.ds(rowbase + r * 16, 16)])
            return 0
        lax.fori_loop(0, n16, rbody, 0)


@jax.jit
def _sc_layer(h, src, dst, e0, e1, e2, e3):
    mesh = plsc.VectorSubcoreMesh(core_axis_name="c", subcore_axis_name="s")
    f = functools.partial(
        pl.kernel,
        mesh=mesh,
        out_type=jax.ShapeDtypeStruct((_NC, _CH, _N, _D), jnp.float32),
        scratch_types=[
            pltpu.VMEM((_K,), jnp.int32),
            pltpu.VMEM((_K,), jnp.int32),
            pltpu.VMEM((_K,), jnp.int32),
            pltpu.VMEM((_K,), jnp.int32),
            pltpu.VMEM((_K, _L), jnp.float32),
            pltpu.VMEM((_K, _L), jnp.float32),
            pltpu.VMEM((_K, _D), jnp.float32),
            pltpu.VMEM((_K, _D), jnp.float32),
            pltpu.VMEM((16, _D), jnp.float32),
            pltpu.SemaphoreType.DMA,
            pltpu.SemaphoreType.DMA,
            pltpu.SemaphoreType.DMA,
            pltpu.SemaphoreType.DMA,
            pltpu.VMEM_SHARED((_N, _D), jnp.float32),
        ],
    )(_sc_layer_kernel)
    return f(h, src, dst, e0, e1, e2, e3)


_GK = 128
_GNCH = _EPW_P // _GK


def _sc_geom_kernel(pos_hbm, src_hbm, dst_hbm, out_hbm,
                    si_a, si_b, di_a, di_b, pa_a, pa_b, pb_a, pb_b,
                    ow_a, ow_b, sem_a0, sem_a1, sem_g0, sem_g1):
    core = lax.axis_index("c")
    sub = lax.axis_index("s")
    wid = sub * _NC + core
    ebase = wid * _EPW_P

    si = (si_a, si_b)
    di = (di_a, di_b)
    pa = (pa_a, pa_b)
    pb = (pb_a, pb_b)
    ow = (ow_a, ow_b)
    sem_as = (sem_a0, sem_a1)
    sem_gs = (sem_g0, sem_g1)

    def issue_small(c, b):
        cb = ebase + c * _GK
        pltpu.async_copy(src_hbm.at[pl.ds(cb, _GK)], si[b], sem_as[b])
        pltpu.async_copy(dst_hbm.at[pl.ds(cb, _GK)], di[b], sem_as[b])

    def wait_small(b):
        pltpu.make_async_copy(src_hbm.at[pl.ds(0, _GK)], si[b],
                              sem_as[b]).wait()
        pltpu.make_async_copy(src_hbm.at[pl.ds(0, _GK)], di[b],
                              sem_as[b]).wait()

    def issue_gather(b):
        pltpu.async_copy(pos_hbm.at[si[b]], pa[b], sem_gs[b])
        pltpu.async_copy(pos_hbm.at[di[b]], pb[b], sem_gs[b])

    def wait_gather(b):
        pltpu.make_async_copy(pos_hbm.at[pl.ds(0, _GK)], pa[b],
                              sem_gs[b]).wait()
        pltpu.make_async_copy(pos_hbm.at[pl.ds(0, _GK)], pb[b],
                              sem_gs[b]).wait()

    def finish(c, b):
        wait_gather(b)

        def body(i, _):
            d = pa[b][i, pl.ds(0, _L)] - pb[b][i, pl.ds(0, _L)]
            ow[b][i] = d * d
            return 0
        lax.fori_loop(0, _GK, body, 0)
        pltpu.sync_copy(ow[b], out_hbm.at[pl.ds(ebase + c * _GK, _GK)])

    issue_small(0, 0)
    wait_small(0)
    issue_gather(0)
    issue_small(1, 1)

    def step(c, cur, oth):
        wait_small(cur)
        issue_gather(cur)
        finish(c - 1, oth)
        issue_small(jnp.minimum(c + 1, _GNCH - 1), oth)

    def pair(p, _):
        step(2 * p + 1, 1, 0)
        step(2 * p + 2, 0, 1)
        return 0
    lax.fori_loop(0, (_GNCH - 1) // 2, pair, 0)
    wait_small(1)
    finish(_GNCH - 1, 0)


@jax.jit
def _sc_geom(pos16, src_p, dst_p):
    mesh = plsc.VectorSubcoreMesh(core_axis_name="c", subcore_axis_name="s")
    f = functools.partial(
        pl.kernel,
        mesh=mesh,
        out_type=jax.ShapeDtypeStruct((_NW * _EPW_P, _L), jnp.float32),
        scratch_types=[
            pltpu.VMEM((_GK,), jnp.int32),
            pltpu.VMEM((_GK,), jnp.int32),
            pltpu.VMEM((_GK,), jnp.int32),
            pltpu.VMEM((_GK,), jnp.int32),
            pltpu.VMEM((_GK, _D), jnp.float32),
            pltpu.VMEM((_GK, _D), jnp.float32),
            pltpu.VMEM((_GK, _D), jnp.float32),
            pltpu.VMEM((_GK, _D), jnp.float32),
            pltpu.VMEM((_GK, _L), jnp.float32),
            pltpu.VMEM((_GK, _L), jnp.float32),
            pltpu.SemaphoreType.DMA,
            pltpu.SemaphoreType.DMA,
            pltpu.SemaphoreType.DMA,
            pltpu.SemaphoreType.DMA,
        ],
    )(_sc_geom_kernel)
    return f(pos16, src_p, dst_p)


_NPW = 320
_EK = 80


def _sc_emb_kernel(emb_hbm, at_hbm, out_hbm, idxv, rowsv, sem):
    core = lax.axis_index("c")
    sub = lax.axis_index("s")
    wid = sub * _NC + core
    base = wid * _NPW
    for c in range(_NPW // _EK):
        pltpu.sync_copy(at_hbm.at[pl.ds(base + c * _EK, _EK)], idxv)
        pltpu.async_copy(emb_hbm.at[idxv], rowsv, sem).wait()
        pltpu.sync_copy(rowsv, out_hbm.at[pl.ds(base + c * _EK, _EK)])


@jax.jit
def _sc_emb(emb, at_p):
    mesh = plsc.VectorSubcoreMesh(core_axis_name="c", subcore_axis_name="s")
    f = functools.partial(
        pl.kernel,
        mesh=mesh,
        out_type=jax.ShapeDtypeStruct((_NW * _NPW, _D), jnp.float32),
        scratch_types=[
            pltpu.VMEM((_EK,), jnp.int32),
            pltpu.VMEM((_EK, _D), jnp.float32),
            pltpu.SemaphoreType.DMA,
        ],
    )(_sc_emb_kernel)
    return f(emb, at_p)


def _node_mlp_body(agg_ref, w1_ref, b1_ref, w2_ref, b2_ref, out_ref):
    acc = None
    for c in range(_CH):
        x = agg_ref[0, c] + agg_ref[1, c]
        h1 = jnp.maximum(
            jnp.dot(x, w1_ref[c], preferred_element_type=jnp.float32) + b1_ref[c],
            0.0,
        )
        y = jnp.dot(h1, w2_ref[c], preferred_element_type=jnp.float32) + b2_ref[c]
        acc = y if acc is None else acc + y
    out_ref[...] = acc


@jax.jit
def _node_mlp(agg, w1, b1, w2, b2):
    return pl.pallas_call(
        _node_mlp_body,
        grid=(_N // _BLK,),
        in_specs=[
            pl.BlockSpec((_NC, _CH, _BLK, _D), lambda i: (0, 0, i, 0)),
            pl.BlockSpec((_CH, _D, _D), lambda i: (0, 0, 0)),
            pl.BlockSpec((_CH, 1, _D), lambda i: (0, 0, 0)),
            pl.BlockSpec((_CH, _D, _D), lambda i: (0, 0, 0)),
            pl.BlockSpec((_CH, 1, _D), lambda i: (0, 0, 0)),
        ],
        out_specs=pl.BlockSpec((_BLK, _D), lambda i: (i, 0)),
        out_shape=jax.ShapeDtypeStruct((_N, _D), jnp.float32),
    )(agg, w1, b1, w2, b2)


def _gn_stats_body(h_ref, bat_ref, s0_ref, s1_ref, c_ref):
    i = pl.program_id(0)
    oh = (bat_ref[...] == lax.broadcasted_iota(jnp.int32, (_BLK, _B), 1)
          ).astype(jnp.float32)
    dn = (((0,), (0,)), ((), ()))
    x = h_ref[...]
    s0 = lax.dot_general(oh, x, dn, preferred_element_type=jnp.float32)
    s1 = lax.dot_general(oh, x * x, dn, preferred_element_type=jnp.float32)
    c = lax.dot_general(oh, jnp.ones((_BLK, _D), jnp.float32), dn,
                        preferred_element_type=jnp.float32)

    @pl.when(i == 0)
    def _():
        s0_ref[...] = jnp.zeros_like(s0_ref)
        s1_ref[...] = jnp.zeros_like(s1_ref)
        c_ref[...] = jnp.zeros_like(c_ref)

    s0_ref[...] += s0
    s1_ref[...] += s1
    c_ref[...] += c


def _gn_apply_body(h_ref, bat_ref, s0_ref, s1_ref, c_ref,
                   w_ref, b_ref, ms_ref, out_ref):
    oh = (bat_ref[...] == lax.broadcasted_iota(jnp.int32, (_BLK, _B), 1)
          ).astype(jnp.float32)
    c = jnp.maximum(c_ref[...], 1.0)
    m = s0_ref[...] / c
    mm = m * ms_ref[...]
    v = s1_ref[...] / c - 2.0 * mm * m + mm * mm
    dn = (((1,), (0,)), ((), ()))
    mrow = lax.dot_general(oh, mm, dn, preferred_element_type=jnp.float32)
    vrow = lax.dot_general(oh, v, dn, preferred_element_type=jnp.float32)
    hc = h_ref[...] - mrow
    out_ref[...] = jnp.tanh(hc / jnp.sqrt(vrow + 1e-5) * w_ref[...]
                            + b_ref[...])


@jax.jit
def _graph_norm(h, bat2d, w, b, ms):
    bspec = pl.BlockSpec((_BLK, 1), lambda i: (i, 0))
    full = pl.BlockSpec((_B, _D), lambda i: (0, 0))
    s0, s1, c = pl.pallas_call(
        _gn_stats_body,
        grid=(_N // _BLK,),
        in_specs=[pl.BlockSpec((_BLK, _D), lambda i: (i, 0)), bspec],
        out_specs=[full, full, full],
        out_shape=[jax.ShapeDtypeStruct((_B, _D), jnp.float32)] * 3,
    )(h, bat2d)
    return pl.pallas_call(
        _gn_apply_body,
        grid=(_N // _BLK,),
        in_specs=[pl.BlockSpec((_BLK, _D), lambda i: (i, 0)), bspec,
                  full, full, full,
                  pl.BlockSpec((1, _D), lambda i: (0, 0)),
                  pl.BlockSpec((1, _D), lambda i: (0, 0)),
                  pl.BlockSpec((1, _D), lambda i: (0, 0))],
        out_specs=pl.BlockSpec((_BLK, _D), lambda i: (i, 0)),
        out_shape=jax.ShapeDtypeStruct((_N, _D), jnp.float32),
    )(h, bat2d, s0, s1, c, w[None, :], b[None, :], ms[None, :])


def _readout_body(h_ref, bat_ref, g_ref, c_ref, out_ref):
    i = pl.program_id(0)
    oh = (bat_ref[...] == lax.broadcasted_iota(jnp.int32, (_BLK, _B), 1)
          ).astype(jnp.float32)
    dn = (((0,), (0,)), ((), ()))
    g = lax.dot_general(oh, h_ref[...], dn, preferred_element_type=jnp.float32)
    c = lax.dot_general(oh, jnp.ones((_BLK, _D), jnp.float32), dn,
                        preferred_element_type=jnp.float32)

    @pl.when(i == 0)
    def _():
        g_ref[...] = jnp.zeros_like(g_ref)
        c_ref[...] = jnp.zeros_like(c_ref)

    g_ref[...] += g
    c_ref[...] += c

    @pl.when(i == _N // _BLK - 1)
    def _():
        gg = g_ref[...] / jnp.maximum(c_ref[...], 1.0)
        out_ref[...] = jnp.mean(gg, axis=1, keepdims=True)


@jax.jit
def _readout(h, bat2d):
    full = pl.BlockSpec((_B, _D), lambda i: (0, 0))
    _, _, out = pl.pallas_call(
        _readout_body,
        grid=(_N // _BLK,),
        in_specs=[pl.BlockSpec((_BLK, _D), lambda i: (i, 0)),
                  pl.BlockSpec((_BLK, 1), lambda i: (i, 0))],
        out_specs=[full, full, pl.BlockSpec((_B, 1), lambda i: (0, 0))],
        out_shape=[jax.ShapeDtypeStruct((_B, _D), jnp.float32),
                   jax.ShapeDtypeStruct((_B, _D), jnp.float32),
                   jax.ShapeDtypeStruct((_B, 1), jnp.float32)],
    )(h, bat2d)
    return out[:, 0]


def _mlp(params, x):
    n = len(params)
    for i, (w, b) in enumerate(params):
        x = x @ w + b
        if i < n - 1:
            x = jax.nn.relu(x)
    return x


def kernel(pos, batch, atom_type, edge_index, params):
    src = edge_index[0]
    dst = edge_index[1]
    pad = _EPW_P - _EPW
    src_p = jnp.pad(src.reshape(_NW, _EPW), ((0, 0), (0, pad))).reshape(-1)
    dst_p = jnp.pad(dst.reshape(_NW, _EPW), ((0, 0), (0, pad))).reshape(-1)
    pos16 = jnp.pad(pos, ((0, 0), (0, _D - 3)))
    d2w = _sc_geom(pos16, src_p, dst_p).reshape(_NW, _EPW_P, _L)[:, :_EPW, :]
    dist = jnp.sqrt(d2w[..., 0] + d2w[..., 1] + d2w[..., 2]
                    + 1e-12).reshape(_E)
    ea = dist[:, None]
    mu = jnp.mean(ea, axis=0)
    var = jnp.var(ea, axis=0)
    edge_attr = (ea - mu) / jnp.sqrt(var + 1e-5) * params['bn_g'] + params['bn_b']
    edge_weight = (_CUTOFF - ea) / _CUTOFF
    at_p = jnp.pad(atom_type, (0, _NW * _NPW - _N))
    h = _sc_emb(params['emb'], at_p)[:_N]
    bat2d = batch[:, None].astype(jnp.int32)
    n_layers = len(params['layers'])
    for li in range(n_layers):
        lp = params['layers'][li]
        e = jax.nn.softmax(_mlp(lp['edge_mlp'], edge_attr), axis=-1)
        e = e * edge_weight
        ee = [jnp.pad(jnp.broadcast_to(e[:, c][:, None], (_E, _L))
                      .reshape(_NW, _EPW, _L), ((0, 0), (0, pad), (0, 0)))
              .reshape(_NW * _EPW_P, _L) for c in range(_CH)]
        agg = _sc_layer(h, src_p, dst_p, ee[0], ee[1], ee[2], ee[3])
        w1 = jnp.stack([lp['node_mlp'][c][0][0] for c in range(_CH)])
        b1 = jnp.stack([lp['node_mlp'][c][0][1] for c in range(_CH)])[:, None, :]
        w2 = jnp.stack([lp['node_mlp'][c][1][0] for c in range(_CH)])
        b2 = jnp.stack([lp['node_mlp'][c][1][1] for c in range(_CH)])[:, None, :]
        h = _node_mlp(agg, w1, b1, w2, b2)
        if li + 1 < n_layers:
            gn = params['norms'][li]
            h = _graph_norm(h, bat2d, gn['weight'], gn['bias'],
                            gn['mean_scale'])
    return _readout(h, bat2d)


# speedup vs baseline: 2.3825x; 1.2578x over previous
import functools

import jax
import jax.numpy as jnp
from jax import lax
from jax.experimental import pallas as pl
from jax.experimental.pallas import tpu as pltpu
from jax.experimental.pallas import tpu_sc as plsc

_N = 10000
_E = 320000
_D = 128
_B = 64
_CH = 4
_CUTOFF = 10.0
_BLK = 400

_NC, _NS, _L = 2, 16, 16
_NW = _NC * _NS
_EPW = _E // _NW
_K = 64
_NCHUNK = 158
_EPW_P = _NCHUNK * _K


def _row_range(s):
    base = s * 624
    n16 = jnp.where(s == _NS - 1, 40, 39)
    return base, n16


def _sc_layer_kernel(h_hbm, src_hbm, dst_hbm, e0_hbm, e1_hbm, e2_hbm, e3_hbm,
                     out_hbm,
                     idx_a, idx_b, dst_a, dst_b, ev_a, ev_b, rows_a, rows_b,
                     zbuf,
                     sem_a0, sem_a1, sem_g0, sem_g1, acc_sh):
    core = lax.axis_index("c")
    sub = lax.axis_index("s")
    wid = sub * _NC + core
    ebase = wid * _EPW_P
    rowbase, n16 = _row_range(sub)

    idx = (idx_a, idx_b)
    dstv = (dst_a, dst_b)
    ev = (ev_a, ev_b)
    rows = (rows_a, rows_b)
    sem_as = (sem_a0, sem_a1)
    sem_gs = (sem_g0, sem_g1)

    def scale_rows(rw, evv):
        def body(i, _):
            es = evv[i]
            for j in range(_D // _L):
                rw[i, pl.ds(j * _L, _L)] = rw[i, pl.ds(j * _L, _L)] * es
            return 0
        lax.fori_loop(0, _K, body, 0)

    for ch, e_hbm in enumerate((e0_hbm, e1_hbm, e2_hbm, e3_hbm)):
        def issue_small(c, b):
            cb = ebase + c * _K
            pltpu.async_copy(src_hbm.at[pl.ds(cb, _K)], idx[b], sem_as[b])
            pltpu.async_copy(dst_hbm.at[pl.ds(cb, _K)], dstv[b], sem_as[b])
            pltpu.async_copy(e_hbm.at[pl.ds(cb, _K)], ev[b], sem_as[b])

        def wait_small(b):
            pltpu.make_async_copy(src_hbm.at[pl.ds(0, _K)], idx[b],
                                  sem_as[b]).wait()
            pltpu.make_async_copy(dst_hbm.at[pl.ds(0, _K)], dstv[b],
                                  sem_as[b]).wait()
            pltpu.make_async_copy(e_hbm.at[pl.ds(0, _K)], ev[b],
                                  sem_as[b]).wait()

        def issue_gather(b):
            pltpu.async_copy(h_hbm.at[idx[b]], rows[b], sem_gs[b])

        def wait_gather(b):
            pltpu.make_async_copy(h_hbm.at[pl.ds(0, _K)], rows[b],
                                  sem_gs[b]).wait()

        def finish(b):
            wait_gather(b)
            scale_rows(rows[b], ev[b])
            pltpu.sync_copy(rows[b], acc_sh.at[dstv[b]], add=True)

        zero16 = jnp.zeros((_L,), jnp.float32)
        for r in range(16):
            for j in range(_D // _L):
                zbuf[r, pl.ds(j * _L, _L)] = zero16

        def zbody(r, _):
            pltpu.sync_copy(zbuf, acc_sh.at[pl.ds(rowbase + r * 16, 16)])
            return 0
        lax.fori_loop(0, n16, zbody, 0)
        plsc.subcore_barrier()

        issue_small(0, 0)
        wait_small(0)
        issue_gather(0)
        issue_small(1, 1)

        def step(c, cur, oth):
            wait_small(cur)
            issue_gather(cur)
            finish(oth)
            issue_small(jnp.minimum(c + 1, _NCHUNK - 1), oth)

        def pair(p, _):
            step(2 * p + 1, 1, 0)
            step(2 * p + 2, 0, 1)
            return 0
        lax.fori_loop(0, (_NCHUNK - 1) // 2, pair, 0)
        wait_small(1)
        finish(0)

        plsc.subcore_barrier()

        def rbody(r, _):
            pltpu.sync_copy(acc_sh.at[pl.ds(rowbase + r * 16, 16)], zbuf)
            pltpu.sync_copy(zbuf, out_hbm.at[core, ch,
                                            pl.ds(rowbase + r * 16, 16)])
            return 0
        lax.fori_loop(0, n16, rbody, 0)


@jax.jit
def _sc_layer(h, src, dst, e0, e1, e2, e3):
    mesh = plsc.VectorSubcoreMesh(core_axis_name="c", subcore_axis_name="s")
    f = functools.partial(
        pl.kernel,
        mesh=mesh,
        out_type=jax.ShapeDtypeStruct((_NC, _CH, _N, _D), jnp.float32),
        scratch_types=[
            pltpu.VMEM((_K,), jnp.int32),
            pltpu.VMEM((_K,), jnp.int32),
            pltpu.VMEM((_K,), jnp.int32),
            pltpu.VMEM((_K,), jnp.int32),
            pltpu.VMEM((_K, _L), jnp.float32),
            pltpu.VMEM((_K, _L), jnp.float32),
            pltpu.VMEM((_K, _D), jnp.float32),
            pltpu.VMEM((_K, _D), jnp.float32),
            pltpu.VMEM((16, _D), jnp.float32),
            pltpu.SemaphoreType.DMA,
            pltpu.SemaphoreType.DMA,
            pltpu.SemaphoreType.DMA,
            pltpu.SemaphoreType.DMA,
            pltpu.VMEM_SHARED((_N, _D), jnp.float32),
        ],
    )(_sc_layer_kernel)
    return f(h, src, dst, e0, e1, e2, e3)


_GK = 128
_GNCH = _EPW_P // _GK


def _sc_geom_kernel(pos_hbm, src_hbm, dst_hbm, out_hbm,
                    si_a, si_b, di_a, di_b, pa_a, pa_b, pb_a, pb_b,
                    ow_a, ow_b, sem_a0, sem_a1, sem_g0, sem_g1):
    core = lax.axis_index("c")
    sub = lax.axis_index("s")
    wid = sub * _NC + core
    ebase = wid * _EPW_P

    si = (si_a, si_b)
    di = (di_a, di_b)
    pa = (pa_a, pa_b)
    pb = (pb_a, pb_b)
    ow = (ow_a, ow_b)
    sem_as = (sem_a0, sem_a1)
    sem_gs = (sem_g0, sem_g1)

    def issue_small(c, b):
        cb = ebase + c * _GK
        pltpu.async_copy(src_hbm.at[pl.ds(cb, _GK)], si[b], sem_as[b])
        pltpu.async_copy(dst_hbm.at[pl.ds(cb, _GK)], di[b], sem_as[b])

    def wait_small(b):
        pltpu.make_async_copy(src_hbm.at[pl.ds(0, _GK)], si[b],
                              sem_as[b]).wait()
        pltpu.make_async_copy(src_hbm.at[pl.ds(0, _GK)], di[b],
                              sem_as[b]).wait()

    def issue_gather(b):
        pltpu.async_copy(pos_hbm.at[si[b]], pa[b], sem_gs[b])
        pltpu.async_copy(pos_hbm.at[di[b]], pb[b], sem_gs[b])

    def wait_gather(b):
        pltpu.make_async_copy(pos_hbm.at[pl.ds(0, _GK)], pa[b],
                              sem_gs[b]).wait()
        pltpu.make_async_copy(pos_hbm.at[pl.ds(0, _GK)], pb[b],
                              sem_gs[b]).wait()

    def finish(c, b):
        wait_gather(b)

        def body(i, _):
            d = pa[b][i, pl.ds(0, _L)] - pb[b][i, pl.ds(0, _L)]
            ow[b][i] = d * d
            return 0
        lax.fori_loop(0, _GK, body, 0)
        pltpu.sync_copy(ow[b], out_hbm.at[pl.ds(ebase + c * _GK, _GK)])

    issue_small(0, 0)
    wait_small(0)
    issue_gather(0)
    issue_small(1, 1)

    def step(c, cur, oth):
        wait_small(cur)
        issue_gather(cur)
        finish(c - 1, oth)
        issue_small(jnp.minimum(c + 1, _GNCH - 1), oth)

    def pair(p, _):
        step(2 * p + 1, 1, 0)
        step(2 * p + 2, 0, 1)
        return 0
    lax.fori_loop(0, (_GNCH - 1) // 2, pair, 0)
    wait_small(1)
    finish(_GNCH - 1, 0)


@jax.jit
def _sc_geom(pos16, src_p, dst_p):
    mesh = plsc.VectorSubcoreMesh(core_axis_name="c", subcore_axis_name="s")
    f = functools.partial(
        pl.kernel,
        mesh=mesh,
        out_type=jax.ShapeDtypeStruct((_NW * _EPW_P, _L), jnp.float32),
        scratch_types=[
            pltpu.VMEM((_GK,), jnp.int32),
            pltpu.VMEM((_GK,), jnp.int32),
            pltpu.VMEM((_GK,), jnp.int32),
            pltpu.VMEM((_GK,), jnp.int32),
            pltpu.VMEM((_GK, _D), jnp.float32),
            pltpu.VMEM((_GK, _D), jnp.float32),
            pltpu.VMEM((_GK, _D), jnp.float32),
            pltpu.VMEM((_GK, _D), jnp.float32),
            pltpu.VMEM((_GK, _L), jnp.float32),
            pltpu.VMEM((_GK, _L), jnp.float32),
            pltpu.SemaphoreType.DMA,
            pltpu.SemaphoreType.DMA,
            pltpu.SemaphoreType.DMA,
            pltpu.SemaphoreType.DMA,
        ],
    )(_sc_geom_kernel)
    return f(pos16, src_p, dst_p)


_NPW = 320
_EK = 80


def _sc_emb_kernel(emb_hbm, at_hbm, out_hbm, idxv, rowsv, sem):
    core = lax.axis_index("c")
    sub = lax.axis_index("s")
    wid = sub * _NC + core
    base = wid * _NPW
    for c in range(_NPW // _EK):
        pltpu.sync_copy(at_hbm.at[pl.ds(base + c * _EK, _EK)], idxv)
        pltpu.async_copy(emb_hbm.at[idxv], rowsv, sem).wait()
        pltpu.sync_copy(rowsv, out_hbm.at[pl.ds(base + c * _EK, _EK)])


@jax.jit
def _sc_emb(emb, at_p):
    mesh = plsc.VectorSubcoreMesh(core_axis_name="c", subcore_axis_name="s")
    f = functools.partial(
        pl.kernel,
        mesh=mesh,
        out_type=jax.ShapeDtypeStruct((_NW * _NPW, _D), jnp.float32),
        scratch_types=[
            pltpu.VMEM((_EK,), jnp.int32),
            pltpu.VMEM((_EK, _D), jnp.float32),
            pltpu.SemaphoreType.DMA,
        ],
    )(_sc_emb_kernel)
    return f(emb, at_p)


def _node_mlp_body(agg_ref, w1_ref, b1_ref, w2_ref, b2_ref, out_ref):
    acc = None
    for c in range(_CH):
        x = agg_ref[0, c] + agg_ref[1, c]
        h1 = jnp.maximum(
            jnp.dot(x, w1_ref[c], preferred_element_type=jnp.float32) + b1_ref[c],
            0.0,
        )
        y = jnp.dot(h1, w2_ref[c], preferred_element_type=jnp.float32) + b2_ref[c]
        acc = y if acc is None else acc + y
    out_ref[...] = acc


@jax.jit
def _node_mlp(agg, w1, b1, w2, b2):
    return pl.pallas_call(
        _node_mlp_body,
        grid=(_N // _BLK,),
        in_specs=[
            pl.BlockSpec((_NC, _CH, _BLK, _D), lambda i: (0, 0, i, 0)),
            pl.BlockSpec((_CH, _D, _D), lambda i: (0, 0, 0)),
            pl.BlockSpec((_CH, 1, _D), lambda i: (0, 0, 0)),
            pl.BlockSpec((_CH, _D, _D), lambda i: (0, 0, 0)),
            pl.BlockSpec((_CH, 1, _D), lambda i: (0, 0, 0)),
        ],
        out_specs=pl.BlockSpec((_BLK, _D), lambda i: (i, 0)),
        out_shape=jax.ShapeDtypeStruct((_N, _D), jnp.float32),
    )(agg, w1, b1, w2, b2)


_EBLK = 1264
_EP = _NW * _EPW_P


def _edge_body(d2_ref, p_ref, p2_ref, o0_ref, o1_ref, o2_ref, o3_ref):
    i = pl.program_id(0)
    x = d2_ref[...]
    dist = jnp.sqrt(x[:, 0] + x[:, 1] + x[:, 2] + 1e-12)
    mu = p_ref[0, 0]
    rsig = p_ref[0, 1]
    bb = p_ref[0, 2]
    ea = (dist - mu) * rsig + bb
    w1 = p2_ref[0, :16]
    b1 = p2_ref[1, :16]
    b2 = p2_ref[2, :4]
    w2 = p2_ref[8:24, :4]
    h1 = jnp.maximum(ea[:, None] * w1[None, :] + b1[None, :], 0.0)
    lg = jnp.dot(h1, w2, preferred_element_type=jnp.float32) + b2[None, :]
    m = jnp.max(lg, axis=1, keepdims=True)
    ex = jnp.exp(lg - m)
    sm = ex / jnp.sum(ex, axis=1, keepdims=True)
    ew = (_CUTOFF - dist) * (1.0 / _CUTOFF)
    row = i * _EBLK + lax.broadcasted_iota(jnp.int32, (_EBLK,), 0)
    mask = ((row % _EPW_P) < _EPW).astype(jnp.float32)
    e4 = sm * (ew * mask)[:, None]
    for c, o_ref in enumerate((o0_ref, o1_ref, o2_ref, o3_ref)):
        o_ref[...] = jnp.broadcast_to(e4[:, c:c + 1], (_EBLK, _L))


@jax.jit
def _edge_stage(d2w_p, p, p2):
    espec = pl.BlockSpec((_EBLK, _L), lambda i: (i, 0))
    return pl.pallas_call(
        _edge_body,
        grid=(_EP // _EBLK,),
        in_specs=[espec,
                  pl.BlockSpec((1, _D), lambda i: (0, 0)),
                  pl.BlockSpec((24, _D), lambda i: (0, 0))],
        out_specs=[espec] * 4,
        out_shape=[jax.ShapeDtypeStruct((_EP, _L), jnp.float32)] * 4,
    )(d2w_p, p, p2)


def _gn_stats_body(h_ref, bat_ref, s0_ref, s1_ref, c_ref):
    i = pl.program_id(0)
    oh = (bat_ref[...] == lax.broadcasted_iota(jnp.int32, (_BLK, _B), 1)
          ).astype(jnp.float32)
    dn = (((0,), (0,)), ((), ()))
    x = h_ref[...]
    s0 = lax.dot_general(oh, x, dn, preferred_element_type=jnp.float32)
    s1 = lax.dot_general(oh, x * x, dn, preferred_element_type=jnp.float32)
    c = lax.dot_general(oh, jnp.ones((_BLK, _D), jnp.float32), dn,
                        preferred_element_type=jnp.float32)

    @pl.when(i == 0)
    def _():
        s0_ref[...] = jnp.zeros_like(s0_ref)
        s1_ref[...] = jnp.zeros_like(s1_ref)
        c_ref[...] = jnp.zeros_like(c_ref)

    s0_ref[...] += s0
    s1_ref[...] += s1
    c_ref[...] += c


def _gn_apply_body(h_ref, bat_ref, s0_ref, s1_ref, c_ref,
                   w_ref, b_ref, ms_ref, out_ref):
    oh = (bat_ref[...] == lax.broadcasted_iota(jnp.int32, (_BLK, _B), 1)
          ).astype(jnp.float32)
    c = jnp.maximum(c_ref[...], 1.0)
    m = s0_ref[...] / c
    mm = m * ms_ref[...]
    v = s1_ref[...] / c - 2.0 * mm * m + mm * mm
    dn = (((1,), (0,)), ((), ()))
    mrow = lax.dot_general(oh, mm, dn, preferred_element_type=jnp.float32)
    vrow = lax.dot_general(oh, v, dn, preferred_element_type=jnp.float32)
    hc = h_ref[...] - mrow
    out_ref[...] = jnp.tanh(hc / jnp.sqrt(vrow + 1e-5) * w_ref[...]
                            + b_ref[...])


@jax.jit
def _graph_norm(h, bat2d, w, b, ms):
    bspec = pl.BlockSpec((_BLK, 1), lambda i: (i, 0))
    full = pl.BlockSpec((_B, _D), lambda i: (0, 0))
    s0, s1, c = pl.pallas_call(
        _gn_stats_body,
        grid=(_N // _BLK,),
        in_specs=[pl.BlockSpec((_BLK, _D), lambda i: (i, 0)), bspec],
        out_specs=[full, full, full],
        out_shape=[jax.ShapeDtypeStruct((_B, _D), jnp.float32)] * 3,
    )(h, bat2d)
    return pl.pallas_call(
        _gn_apply_body,
        grid=(_N // _BLK,),
        in_specs=[pl.BlockSpec((_BLK, _D), lambda i: (i, 0)), bspec,
                  full, full, full,
                  pl.BlockSpec((1, _D), lambda i: (0, 0)),
                  pl.BlockSpec((1, _D), lambda i: (0, 0)),
                  pl.BlockSpec((1, _D), lambda i: (0, 0))],
        out_specs=pl.BlockSpec((_BLK, _D), lambda i: (i, 0)),
        out_shape=jax.ShapeDtypeStruct((_N, _D), jnp.float32),
    )(h, bat2d, s0, s1, c, w[None, :], b[None, :], ms[None, :])


def _readout_body(h_ref, bat_ref, g_ref, c_ref, out_ref):
    i = pl.program_id(0)
    oh = (bat_ref[...] == lax.broadcasted_iota(jnp.int32, (_BLK, _B), 1)
          ).astype(jnp.float32)
    dn = (((0,), (0,)), ((), ()))
    g = lax.dot_general(oh, h_ref[...], dn, preferred_element_type=jnp.float32)
    c = lax.dot_general(oh, jnp.ones((_BLK, _D), jnp.float32), dn,
                        preferred_element_type=jnp.float32)

    @pl.when(i == 0)
    def _():
        g_ref[...] = jnp.zeros_like(g_ref)
        c_ref[...] = jnp.zeros_like(c_ref)

    g_ref[...] += g
    c_ref[...] += c

    @pl.when(i == _N // _BLK - 1)
    def _():
        gg = g_ref[...] / jnp.maximum(c_ref[...], 1.0)
        out_ref[...] = jnp.mean(gg, axis=1, keepdims=True)


@jax.jit
def _readout(h, bat2d):
    full = pl.BlockSpec((_B, _D), lambda i: (0, 0))
    _, _, out = pl.pallas_call(
        _readout_body,
        grid=(_N // _BLK,),
        in_specs=[pl.BlockSpec((_BLK, _D), lambda i: (i, 0)),
                  pl.BlockSpec((_BLK, 1), lambda i: (i, 0))],
        out_specs=[full, full, pl.BlockSpec((_B, 1), lambda i: (0, 0))],
        out_shape=[jax.ShapeDtypeStruct((_B, _D), jnp.float32),
                   jax.ShapeDtypeStruct((_B, _D), jnp.float32),
                   jax.ShapeDtypeStruct((_B, 1), jnp.float32)],
    )(h, bat2d)
    return out[:, 0]


def _mlp(params, x):
    n = len(params)
    for i, (w, b) in enumerate(params):
        x = x @ w + b
        if i < n - 1:
            x = jax.nn.relu(x)
    return x


def kernel(pos, batch, atom_type, edge_index, params):
    src = edge_index[0]
    dst = edge_index[1]
    pad = _EPW_P - _EPW
    src_p = jnp.pad(src.reshape(_NW, _EPW), ((0, 0), (0, pad))).reshape(-1)
    dst_p = jnp.pad(dst.reshape(_NW, _EPW), ((0, 0), (0, pad))).reshape(-1)
    pos16 = jnp.pad(pos, ((0, 0), (0, _D - 3)))
    d2w_p = _sc_geom(pos16, src_p, dst_p)
    dist_p = jnp.sqrt(d2w_p[:, 0] + d2w_p[:, 1] + d2w_p[:, 2] + 1e-12)
    emask = (jnp.arange(_EP) % _EPW_P) < _EPW
    dm = jnp.where(emask, dist_p, 0.0)
    mu = jnp.sum(dm) / _E
    dc = jnp.where(emask, dist_p - mu, 0.0)
    var = jnp.sum(dc * dc) / _E
    rsig = params['bn_g'][0] / jnp.sqrt(var + 1e-5)
    pvec = (jnp.zeros((1, _D), jnp.float32)
            .at[0, 0].set(mu).at[0, 1].set(rsig)
            .at[0, 2].set(params['bn_b'][0]))
    at_p = jnp.pad(atom_type, (0, _NW * _NPW - _N))
    h = _sc_emb(params['emb'], at_p)[:_N]
    bat2d = batch[:, None].astype(jnp.int32)
    n_layers = len(params['layers'])
    for li in range(n_layers):
        lp = params['layers'][li]
        (w1e, b1e), (w2e, b2e) = lp['edge_mlp']
        p2 = (jnp.zeros((24, _D), jnp.float32)
              .at[0, :16].set(w1e[0]).at[1, :16].set(b1e)
              .at[2, :4].set(b2e).at[8:24, :4].set(w2e))
        ee = _edge_stage(d2w_p, pvec, p2)
        agg = _sc_layer(h, src_p, dst_p, ee[0], ee[1], ee[2], ee[3])
        w1 = jnp.stack([lp['node_mlp'][c][0][0] for c in range(_CH)])
        b1 = jnp.stack([lp['node_mlp'][c][0][1] for c in range(_CH)])[:, None, :]
        w2 = jnp.stack([lp['node_mlp'][c][1][0] for c in range(_CH)])
        b2 = jnp.stack([lp['node_mlp'][c][1][1] for c in range(_CH)])[:, None, :]
        h = _node_mlp(agg, w1, b1, w2, b2)
        if li + 1 < n_layers:
            gn = params['norms'][li]
            h = _graph_norm(h, bat2d, gn['weight'], gn['bias'],
                            gn['mean_scale'])
    return _readout(h, bat2d)


# speedup vs baseline: 2.6051x; 1.0935x over previous
import functools

import jax
import jax.numpy as jnp
from jax import lax
from jax.experimental import pallas as pl
from jax.experimental.pallas import tpu as pltpu
from jax.experimental.pallas import tpu_sc as plsc

_N = 10000
_E = 320000
_D = 128
_B = 64
_CH = 4
_CUTOFF = 10.0
_BLK = 400

_NC, _NS, _L = 2, 16, 16
_NW = _NC * _NS
_EPW = _E // _NW
_K = 64
_NCHUNK = 158
_EPW_P = _NCHUNK * _K


def _row_range(s):
    base = s * 624
    n16 = jnp.where(s == _NS - 1, 40, 39)
    return base, n16


def _sc_layer_kernel(h_hbm, src_hbm, dst_hbm, e0_hbm, e1_hbm, e2_hbm, e3_hbm,
                     out_hbm,
                     idx_a, idx_b, dst_a, dst_b, ev_a, ev_b, rows_a, rows_b,
                     zbuf,
                     sem_a0, sem_a1, sem_g0, sem_g1, sem_s0, sem_s1, acc_sh):
    core = lax.axis_index("c")
    sub = lax.axis_index("s")
    wid = sub * _NC + core
    ebase = wid * _EPW_P
    rowbase, n16 = _row_range(sub)

    idx = (idx_a, idx_b)
    dstv = (dst_a, dst_b)
    ev = (ev_a, ev_b)
    rows = (rows_a, rows_b)
    sem_as = (sem_a0, sem_a1)
    sem_gs = (sem_g0, sem_g1)
    sem_ss = (sem_s0, sem_s1)

    def scale_rows(rw, evv):
        def body(i, _):
            es = evv[i]
            for j in range(_D // _L):
                rw[i, pl.ds(j * _L, _L)] = rw[i, pl.ds(j * _L, _L)] * es
            return 0
        lax.fori_loop(0, _K, body, 0)

    for ch, e_hbm in enumerate((e0_hbm, e1_hbm, e2_hbm, e3_hbm)):
        def issue_small(c, b):
            cb = ebase + c * _K
            pltpu.async_copy(src_hbm.at[pl.ds(cb, _K)], idx[b], sem_as[b])
            pltpu.async_copy(dst_hbm.at[pl.ds(cb, _K)], dstv[b], sem_as[b])
            pltpu.async_copy(e_hbm.at[pl.ds(cb, _K)], ev[b], sem_as[b])

        def wait_small(b):
            pltpu.make_async_copy(src_hbm.at[pl.ds(0, _K)], idx[b],
                                  sem_as[b]).wait()
            pltpu.make_async_copy(dst_hbm.at[pl.ds(0, _K)], dstv[b],
                                  sem_as[b]).wait()
            pltpu.make_async_copy(e_hbm.at[pl.ds(0, _K)], ev[b],
                                  sem_as[b]).wait()

        def issue_gather(b):
            pltpu.async_copy(h_hbm.at[idx[b]], rows[b], sem_gs[b])

        def wait_gather(b):
            pltpu.make_async_copy(h_hbm.at[pl.ds(0, _K)], rows[b],
                                  sem_gs[b]).wait()

        def wait_scatter(b):
            pltpu.make_async_copy(h_hbm.at[pl.ds(0, _K)], rows[b],
                                  sem_ss[b]).wait()

        def finish(b):
            wait_gather(b)
            scale_rows(rows[b], ev[b])
            pltpu.async_copy(rows[b], acc_sh.at[dstv[b]], sem_ss[b],
                             add=True)

        zero16 = jnp.zeros((_L,), jnp.float32)
        for r in range(16):
            for j in range(_D // _L):
                zbuf[r, pl.ds(j * _L, _L)] = zero16

        def zbody(r, _):
            pltpu.sync_copy(zbuf, acc_sh.at[pl.ds(rowbase + r * 16, 16)])
            return 0
        lax.fori_loop(0, n16, zbody, 0)
        plsc.subcore_barrier()

        issue_small(0, 0)
        wait_small(0)
        issue_gather(0)
        issue_small(1, 1)

        def step(c, cur, oth, drain):
            wait_small(cur)
            if drain:
                wait_scatter(cur)
            issue_gather(cur)
            finish(oth)
            issue_small(jnp.minimum(c + 1, _NCHUNK - 1), oth)

        step(1, 1, 0, False)
        step(2, 0, 1, True)

        def pair(p, _):
            step(2 * p + 1, 1, 0, True)
            step(2 * p + 2, 0, 1, True)
            return 0
        lax.fori_loop(1, (_NCHUNK - 1) // 2, pair, 0)
        wait_small(1)
        finish(0)
        wait_scatter(0)
        wait_scatter(1)

        plsc.subcore_barrier()

        def rbody(r, _):
            pltpu.sync_copy(acc_sh.at[pl.ds(rowbase + r * 16, 16)], zbuf)
            pltpu.sync_copy(zbuf, out_hbm.at[core, ch,
                                            pl.ds(rowbase + r * 16, 16)])
            return 0
        lax.fori_loop(0, n16, rbody, 0)


@jax.jit
def _sc_layer(h, src, dst, e0, e1, e2, e3):
    mesh = plsc.VectorSubcoreMesh(core_axis_name="c", subcore_axis_name="s")
    f = functools.partial(
        pl.kernel,
        mesh=mesh,
        out_type=jax.ShapeDtypeStruct((_NC, _CH, _N, _D), jnp.float32),
        scratch_types=[
            pltpu.VMEM((_K,), jnp.int32),
            pltpu.VMEM((_K,), jnp.int32),
            pltpu.VMEM((_K,), jnp.int32),
            pltpu.VMEM((_K,), jnp.int32),
            pltpu.VMEM((_K, _L), jnp.float32),
            pltpu.VMEM((_K, _L), jnp.float32),
            pltpu.VMEM((_K, _D), jnp.float32),
            pltpu.VMEM((_K, _D), jnp.float32),
            pltpu.VMEM((16, _D), jnp.float32),
            pltpu.SemaphoreType.DMA,
            pltpu.SemaphoreType.DMA,
            pltpu.SemaphoreType.DMA,
            pltpu.SemaphoreType.DMA,
            pltpu.SemaphoreType.DMA,
            pltpu.SemaphoreType.DMA,
            pltpu.VMEM_SHARED((_N, _D), jnp.float32),
        ],
    )(_sc_layer_kernel)
    return f(h, src, dst, e0, e1, e2, e3)


_GK = 128
_GNCH = _EPW_P // _GK


def _sc_geom_kernel(pos_hbm, src_hbm, dst_hbm, out_hbm,
                    si_a, si_b, di_a, di_b, pa_a, pa_b, pb_a, pb_b,
                    ow_a, ow_b, sem_a0, sem_a1, sem_g0, sem_g1):
    core = lax.axis_index("c")
    sub = lax.axis_index("s")
    wid = sub * _NC + core
    ebase = wid * _EPW_P

    si = (si_a, si_b)
    di = (di_a, di_b)
    pa = (pa_a, pa_b)
    pb = (pb_a, pb_b)
    ow = (ow_a, ow_b)
    sem_as = (sem_a0, sem_a1)
    sem_gs = (sem_g0, sem_g1)

    def issue_small(c, b):
        cb = ebase + c * _GK
        pltpu.async_copy(src_hbm.at[pl.ds(cb, _GK)], si[b], sem_as[b])
        pltpu.async_copy(dst_hbm.at[pl.ds(cb, _GK)], di[b], sem_as[b])

    def wait_small(b):
        pltpu.make_async_copy(src_hbm.at[pl.ds(0, _GK)], si[b],
                              sem_as[b]).wait()
        pltpu.make_async_copy(src_hbm.at[pl.ds(0, _GK)], di[b],
                              sem_as[b]).wait()

    def issue_gather(b):
        pltpu.async_copy(pos_hbm.at[si[b]], pa[b], sem_gs[b])
        pltpu.async_copy(pos_hbm.at[di[b]], pb[b], sem_gs[b])

    def wait_gather(b):
        pltpu.make_async_copy(pos_hbm.at[pl.ds(0, _GK)], pa[b],
                              sem_gs[b]).wait()
        pltpu.make_async_copy(pos_hbm.at[pl.ds(0, _GK)], pb[b],
                              sem_gs[b]).wait()

    def finish(c, b):
        wait_gather(b)

        def body(i, _):
            d = pa[b][i, pl.ds(0, _L)] - pb[b][i, pl.ds(0, _L)]
            ow[b][i] = d * d
            return 0
        lax.fori_loop(0, _GK, body, 0)
        pltpu.sync_copy(ow[b], out_hbm.at[pl.ds(ebase + c * _GK, _GK)])

    issue_small(0, 0)
    wait_small(0)
    issue_gather(0)
    issue_small(1, 1)

    def step(c, cur, oth):
        wait_small(cur)
        issue_gather(cur)
        finish(c - 1, oth)
        issue_small(jnp.minimum(c + 1, _GNCH - 1), oth)

    def pair(p, _):
        step(2 * p + 1, 1, 0)
        step(2 * p + 2, 0, 1)
        return 0
    lax.fori_loop(0, (_GNCH - 1) // 2, pair, 0)
    wait_small(1)
    finish(_GNCH - 1, 0)


@jax.jit
def _sc_geom(pos16, src_p, dst_p):
    mesh = plsc.VectorSubcoreMesh(core_axis_name="c", subcore_axis_name="s")
    f = functools.partial(
        pl.kernel,
        mesh=mesh,
        out_type=jax.ShapeDtypeStruct((_NW * _EPW_P, _L), jnp.float32),
        scratch_types=[
            pltpu.VMEM((_GK,), jnp.int32),
            pltpu.VMEM((_GK,), jnp.int32),
            pltpu.VMEM((_GK,), jnp.int32),
            pltpu.VMEM((_GK,), jnp.int32),
            pltpu.VMEM((_GK, _D), jnp.float32),
            pltpu.VMEM((_GK, _D), jnp.float32),
            pltpu.VMEM((_GK, _D), jnp.float32),
            pltpu.VMEM((_GK, _D), jnp.float32),
            pltpu.VMEM((_GK, _L), jnp.float32),
            pltpu.VMEM((_GK, _L), jnp.float32),
            pltpu.SemaphoreType.DMA,
            pltpu.SemaphoreType.DMA,
            pltpu.SemaphoreType.DMA,
            pltpu.SemaphoreType.DMA,
        ],
    )(_sc_geom_kernel)
    return f(pos16, src_p, dst_p)


_NPW = 320
_EK = 80


def _sc_emb_kernel(emb_hbm, at_hbm, out_hbm, idxv, rowsv, sem):
    core = lax.axis_index("c")
    sub = lax.axis_index("s")
    wid = sub * _NC + core
    base = wid * _NPW
    for c in range(_NPW // _EK):
        pltpu.sync_copy(at_hbm.at[pl.ds(base + c * _EK, _EK)], idxv)
        pltpu.async_copy(emb_hbm.at[idxv], rowsv, sem).wait()
        pltpu.sync_copy(rowsv, out_hbm.at[pl.ds(base + c * _EK, _EK)])


@jax.jit
def _sc_emb(emb, at_p):
    mesh = plsc.VectorSubcoreMesh(core_axis_name="c", subcore_axis_name="s")
    f = functools.partial(
        pl.kernel,
        mesh=mesh,
        out_type=jax.ShapeDtypeStruct((_NW * _NPW, _D), jnp.float32),
        scratch_types=[
            pltpu.VMEM((_EK,), jnp.int32),
            pltpu.VMEM((_EK, _D), jnp.float32),
            pltpu.SemaphoreType.DMA,
        ],
    )(_sc_emb_kernel)
    return f(emb, at_p)


def _node_mlp_body(agg_ref, w1_ref, b1_ref, w2_ref, b2_ref, out_ref):
    acc = None
    for c in range(_CH):
        x = agg_ref[0, c] + agg_ref[1, c]
        h1 = jnp.maximum(
            jnp.dot(x, w1_ref[c], preferred_element_type=jnp.float32) + b1_ref[c],
            0.0,
        )
        y = jnp.dot(h1, w2_ref[c], preferred_element_type=jnp.float32) + b2_ref[c]
        acc = y if acc is None else acc + y
    out_ref[...] = acc


@jax.jit
def _node_mlp(agg, w1, b1, w2, b2):
    return pl.pallas_call(
        _node_mlp_body,
        grid=(_N // _BLK,),
        in_specs=[
            pl.BlockSpec((_NC, _CH, _BLK, _D), lambda i: (0, 0, i, 0)),
            pl.BlockSpec((_CH, _D, _D), lambda i: (0, 0, 0)),
            pl.BlockSpec((_CH, 1, _D), lambda i: (0, 0, 0)),
            pl.BlockSpec((_CH, _D, _D), lambda i: (0, 0, 0)),
            pl.BlockSpec((_CH, 1, _D), lambda i: (0, 0, 0)),
        ],
        out_specs=pl.BlockSpec((_BLK, _D), lambda i: (i, 0)),
        out_shape=jax.ShapeDtypeStruct((_N, _D), jnp.float32),
    )(agg, w1, b1, w2, b2)


_EBLK = 1264
_EP = _NW * _EPW_P


def _edge_body(d2_ref, p_ref, p2_ref, o0_ref, o1_ref, o2_ref, o3_ref):
    i = pl.program_id(0)
    x = d2_ref[...]
    dist = jnp.sqrt(x[:, 0] + x[:, 1] + x[:, 2] + 1e-12)
    mu = p_ref[0, 0]
    rsig = p_ref[0, 1]
    bb = p_ref[0, 2]
    ea = (dist - mu) * rsig + bb
    w1 = p2_ref[0, :16]
    b1 = p2_ref[1, :16]
    b2 = p2_ref[2, :4]
    w2 = p2_ref[8:24, :4]
    h1 = jnp.maximum(ea[:, None] * w1[None, :] + b1[None, :], 0.0)
    lg = jnp.dot(h1, w2, preferred_element_type=jnp.float32) + b2[None, :]
    m = jnp.max(lg, axis=1, keepdims=True)
    ex = jnp.exp(lg - m)
    sm = ex / jnp.sum(ex, axis=1, keepdims=True)
    ew = (_CUTOFF - dist) * (1.0 / _CUTOFF)
    row = i * _EBLK + lax.broadcasted_iota(jnp.int32, (_EBLK,), 0)
    mask = ((row % _EPW_P) < _EPW).astype(jnp.float32)
    e4 = sm * (ew * mask)[:, None]
    for c, o_ref in enumerate((o0_ref, o1_ref, o2_ref, o3_ref)):
        o_ref[...] = jnp.broadcast_to(e4[:, c:c + 1], (_EBLK, _L))


@jax.jit
def _edge_stage(d2w_p, p, p2):
    espec = pl.BlockSpec((_EBLK, _L), lambda i: (i, 0))
    return pl.pallas_call(
        _edge_body,
        grid=(_EP // _EBLK,),
        in_specs=[espec,
                  pl.BlockSpec((1, _D), lambda i: (0, 0)),
                  pl.BlockSpec((24, _D), lambda i: (0, 0))],
        out_specs=[espec] * 4,
        out_shape=[jax.ShapeDtypeStruct((_EP, _L), jnp.float32)] * 4,
    )(d2w_p, p, p2)


def _gn_stats_body(h_ref, bat_ref, s0_ref, s1_ref, c_ref):
    i = pl.program_id(0)
    oh = (bat_ref[...] == lax.broadcasted_iota(jnp.int32, (_BLK, _B), 1)
          ).astype(jnp.float32)
    dn = (((0,), (0,)), ((), ()))
    x = h_ref[...]
    s0 = lax.dot_general(oh, x, dn, preferred_element_type=jnp.float32)
    s1 = lax.dot_general(oh, x * x, dn, preferred_element_type=jnp.float32)
    c = lax.dot_general(oh, jnp.ones((_BLK, _D), jnp.float32), dn,
                        preferred_element_type=jnp.float32)

    @pl.when(i == 0)
    def _():
        s0_ref[...] = jnp.zeros_like(s0_ref)
        s1_ref[...] = jnp.zeros_like(s1_ref)
        c_ref[...] = jnp.zeros_like(c_ref)

    s0_ref[...] += s0
    s1_ref[...] += s1
    c_ref[...] += c


def _gn_apply_body(h_ref, bat_ref, s0_ref, s1_ref, c_ref,
                   w_ref, b_ref, ms_ref, out_ref):
    oh = (bat_ref[...] == lax.broadcasted_iota(jnp.int32, (_BLK, _B), 1)
          ).astype(jnp.float32)
    c = jnp.maximum(c_ref[...], 1.0)
    m = s0_ref[...] / c
    mm = m * ms_ref[...]
    v = s1_ref[...] / c - 2.0 * mm * m + mm * mm
    dn = (((1,), (0,)), ((), ()))
    mrow = lax.dot_general(oh, mm, dn, preferred_element_type=jnp.float32)
    vrow = lax.dot_general(oh, v, dn, preferred_element_type=jnp.float32)
    hc = h_ref[...] - mrow
    out_ref[...] = jnp.tanh(hc / jnp.sqrt(vrow + 1e-5) * w_ref[...]
                            + b_ref[...])


@jax.jit
def _graph_norm(h, bat2d, w, b, ms):
    bspec = pl.BlockSpec((_BLK, 1), lambda i: (i, 0))
    full = pl.BlockSpec((_B, _D), lambda i: (0, 0))
    s0, s1, c = pl.pallas_call(
        _gn_stats_body,
        grid=(_N // _BLK,),
        in_specs=[pl.BlockSpec((_BLK, _D), lambda i: (i, 0)), bspec],
        out_specs=[full, full, full],
        out_shape=[jax.ShapeDtypeStruct((_B, _D), jnp.float32)] * 3,
    )(h, bat2d)
    return pl.pallas_call(
        _gn_apply_body,
        grid=(_N // _BLK,),
        in_specs=[pl.BlockSpec((_BLK, _D), lambda i: (i, 0)), bspec,
                  full, full, full,
                  pl.BlockSpec((1, _D), lambda i: (0, 0)),
                  pl.BlockSpec((1, _D), lambda i: (0, 0)),
                  pl.BlockSpec((1, _D), lambda i: (0, 0))],
        out_specs=pl.BlockSpec((_BLK, _D), lambda i: (i, 0)),
        out_shape=jax.ShapeDtypeStruct((_N, _D), jnp.float32),
    )(h, bat2d, s0, s1, c, w[None, :], b[None, :], ms[None, :])


def _readout_body(h_ref, bat_ref, g_ref, c_ref, out_ref):
    i = pl.program_id(0)
    oh = (bat_ref[...] == lax.broadcasted_iota(jnp.int32, (_BLK, _B), 1)
          ).astype(jnp.float32)
    dn = (((0,), (0,)), ((), ()))
    g = lax.dot_general(oh, h_ref[...], dn, preferred_element_type=jnp.float32)
    c = lax.dot_general(oh, jnp.ones((_BLK, _D), jnp.float32), dn,
                        preferred_element_type=jnp.float32)

    @pl.when(i == 0)
    def _():
        g_ref[...] = jnp.zeros_like(g_ref)
        c_ref[...] = jnp.zeros_like(c_ref)

    g_ref[...] += g
    c_ref[...] += c

    @pl.when(i == _N // _BLK - 1)
    def _():
        gg = g_ref[...] / jnp.maximum(c_ref[...], 1.0)
        out_ref[...] = jnp.mean(gg, axis=1, keepdims=True)


@jax.jit
def _readout(h, bat2d):
    full = pl.BlockSpec((_B, _D), lambda i: (0, 0))
    _, _, out = pl.pallas_call(
        _readout_body,
        grid=(_N // _BLK,),
        in_specs=[pl.BlockSpec((_BLK, _D), lambda i: (i, 0)),
                  pl.BlockSpec((_BLK, 1), lambda i: (i, 0))],
        out_specs=[full, full, pl.BlockSpec((_B, 1), lambda i: (0, 0))],
        out_shape=[jax.ShapeDtypeStruct((_B, _D), jnp.float32),
                   jax.ShapeDtypeStruct((_B, _D), jnp.float32),
                   jax.ShapeDtypeStruct((_B, 1), jnp.float32)],
    )(h, bat2d)
    return out[:, 0]


def _mlp(params, x):
    n = len(params)
    for i, (w, b) in enumerate(params):
        x = x @ w + b
        if i < n - 1:
            x = jax.nn.relu(x)
    return x


def kernel(pos, batch, atom_type, edge_index, params):
    src = edge_index[0]
    dst = edge_index[1]
    pad = _EPW_P - _EPW
    src_p = jnp.pad(src.reshape(_NW, _EPW), ((0, 0), (0, pad))).reshape(-1)
    dst_p = jnp.pad(dst.reshape(_NW, _EPW), ((0, 0), (0, pad))).reshape(-1)
    pos16 = jnp.pad(pos, ((0, 0), (0, _D - 3)))
    d2w_p = _sc_geom(pos16, src_p, dst_p)
    dist_p = jnp.sqrt(d2w_p[:, 0] + d2w_p[:, 1] + d2w_p[:, 2] + 1e-12)
    emask = (jnp.arange(_EP) % _EPW_P) < _EPW
    dm = jnp.where(emask, dist_p, 0.0)
    mu = jnp.sum(dm) / _E
    dc = jnp.where(emask, dist_p - mu, 0.0)
    var = jnp.sum(dc * dc) / _E
    rsig = params['bn_g'][0] / jnp.sqrt(var + 1e-5)
    pvec = (jnp.zeros((1, _D), jnp.float32)
            .at[0, 0].set(mu).at[0, 1].set(rsig)
            .at[0, 2].set(params['bn_b'][0]))
    at_p = jnp.pad(atom_type, (0, _NW * _NPW - _N))
    h = _sc_emb(params['emb'], at_p)[:_N]
    bat2d = batch[:, None].astype(jnp.int32)
    n_layers = len(params['layers'])
    for li in range(n_layers):
        lp = params['layers'][li]
        (w1e, b1e), (w2e, b2e) = lp['edge_mlp']
        p2 = (jnp.zeros((24, _D), jnp.float32)
              .at[0, :16].set(w1e[0]).at[1, :16].set(b1e)
              .at[2, :4].set(b2e).at[8:24, :4].set(w2e))
        ee = _edge_stage(d2w_p, pvec, p2)
        agg = _sc_layer(h, src_p, dst_p, ee[0], ee[1], ee[2], ee[3])
        w1 = jnp.stack([lp['node_mlp'][c][0][0] for c in range(_CH)])
        b1 = jnp.stack([lp['node_mlp'][c][0][1] for c in range(_CH)])[:, None, :]
        w2 = jnp.stack([lp['node_mlp'][c][1][0] for c in range(_CH)])
        b2 = jnp.stack([lp['node_mlp'][c][1][1] for c in range(_CH)])[:, None, :]
        h = _node_mlp(agg, w1, b1, w2, b2)
        if li + 1 < n_layers:
            gn = params['norms'][li]
            h = _graph_norm(h, bat2d, gn['weight'], gn['bias'],
                            gn['mean_scale'])
    return _readout(h, bat2d)
